# baseline (speedup 1.0000x reference)
import functools

import jax
import jax.numpy as jnp
from jax import lax
from jax.experimental import pallas as pl
from jax.experimental.pallas import tpu as pltpu
from jax.experimental.pallas import tpu_sc as plsc

HID = 128
NT = 4
D1 = 2 * HID
NNODES = 10000
NEDGES = 320000

NC = 2
NS = 16
NW = NC * NS

EPW = NEDGES // NW
GK = 80
GCH = EPW // GK
SCC = 80
SCH = EPW // SCC
SCG = SCC // 16

NROWS = NNODES * NT
NROWSP = 40064
NACC = 7

EB = 1280
NB = 1000


def _silu(x):
    return x * (0.5 * jnp.tanh(0.5 * x) + 0.5)


def _proj_body(h_ref, wa_ref, wb_ref, a_ref, b_ref):
    for t in range(NT):
        ht = h_ref[:, t, :]
        a_ref[:, t, :] = jnp.dot(
            ht, wa_ref[t], preferred_element_type=jnp.float32).astype(jnp.bfloat16)
        b_ref[:, t, :] = jnp.dot(
            ht, wb_ref[t], preferred_element_type=jnp.float32).astype(jnp.bfloat16)


def _proj(h, wa, wb):
    n = h.shape[0]
    grid = n // NB
    return pl.pallas_call(
        _proj_body,
        grid=(grid,),
        in_specs=[
            pl.BlockSpec((NB, NT, HID), lambda i: (i, 0, 0)),
            pl.BlockSpec((NT, HID, D1), lambda i: (0, 0, 0)),
            pl.BlockSpec((NT, HID, D1), lambda i: (0, 0, 0)),
        ],
        out_specs=[
            pl.BlockSpec((NB, NT, D1), lambda i: (i, 0, 0)),
            pl.BlockSpec((NB, NT, D1), lambda i: (i, 0, 0)),
        ],
        out_shape=[
            jax.ShapeDtypeStruct((n, NT, D1), jnp.bfloat16),
            jax.ShapeDtypeStruct((n, NT, D1), jnp.bfloat16),
        ],
    )(h, wa, wb)


def _gather_kernel(a_hbm, b_hbm, gr_hbm, gc_hbm, ga_hbm, gb_hbm,
                   ir_v, ic_v, ra_v, rb_v, sa, sb):
    cid = lax.axis_index("c")
    sid = lax.axis_index("s")
    wid = sid * NC + cid
    base_w = wid * EPW

    def body(i, carry):
        base = base_w + i * GK
        pltpu.sync_copy(gr_hbm.at[pl.ds(base, GK)], ir_v)
        pltpu.sync_copy(gc_hbm.at[pl.ds(base, GK)], ic_v)
        da = pltpu.async_copy(a_hbm.at[ir_v], ra_v, sa)
        db = pltpu.async_copy(b_hbm.at[ic_v], rb_v, sb)
        da.wait()
        db.wait()
        pltpu.sync_copy(ra_v, ga_hbm.at[pl.ds(base, GK)])
        pltpu.sync_copy(rb_v, gb_hbm.at[pl.ds(base, GK)])
        return carry

    lax.fori_loop(0, GCH, body, 0)


def _gather(a2, b2, gr, gc):
    mesh = plsc.VectorSubcoreMesh(core_axis_name="c", subcore_axis_name="s",
                                  num_cores=NC, num_subcores=NS)
    fn = functools.partial(
        pl.kernel,
        out_type=(jax.ShapeDtypeStruct((NEDGES, D1 // 2), jnp.int32),
                  jax.ShapeDtypeStruct((NEDGES, D1 // 2), jnp.int32)),
        mesh=mesh,
        scratch_types=[
            pltpu.VMEM((GK,), jnp.int32),
            pltpu.VMEM((GK,), jnp.int32),
            pltpu.VMEM((GK, D1 // 2), jnp.int32),
            pltpu.VMEM((GK, D1 // 2), jnp.int32),
            pltpu.SemaphoreType.DMA,
            pltpu.SemaphoreType.DMA,
        ],
    )(_gather_kernel)
    return fn(a2, b2, gr, gc)


def _mlp_body(ga_ref, gb_ref, attr_ref, mask_ref, el_ref,
              w1c_ref, b1_ref, w2_ref, b2_ref, w3_ref, b3_ref, w4_ref, b4_ref,
              q_ref):
    g = ga_ref[:].astype(jnp.float32) + gb_ref[:].astype(jnp.float32)
    a = attr_ref[:]
    acc = jnp.zeros((g.shape[0], 1), jnp.float32)
    for t in range(NT):
        l1 = g + a * w1c_ref[t] + b1_ref[t]
        x = _silu(l1).astype(jnp.bfloat16)
        x = _silu(jnp.dot(x, w2_ref[t], preferred_element_type=jnp.float32)
                  + b2_ref[t]).astype(jnp.bfloat16)
        x = _silu(jnp.dot(x, w3_ref[t], preferred_element_type=jnp.float32)
                  + b3_ref[t])
        s = jnp.sum(x * w4_ref[t], axis=1, keepdims=True) + b4_ref[0, t]
        acc = acc + mask_ref[:, t:t + 1] * s
    q_ref[:] = acc / el_ref[:]


def _edge_mlp(ga, gb, attr, maskf, el, w1c, b1, w2, b2, w3, b3, w4, b4):
    grid = NEDGES // EB
    full = lambda shape: pl.BlockSpec(shape, lambda i: tuple(0 for _ in shape))
    return pl.pallas_call(
        _mlp_body,
        grid=(grid,),
        in_specs=[
            pl.BlockSpec((EB, D1), lambda i: (i, 0)),
            pl.BlockSpec((EB, D1), lambda i: (i, 0)),
            pl.BlockSpec((EB, 1), lambda i: (i, 0)),
            pl.BlockSpec((EB, NT), lambda i: (i, 0)),
            pl.BlockSpec((EB, 1), lambda i: (i, 0)),
            full((NT, D1)),
            full((NT, D1)),
            full((NT, D1, HID)),
            full((NT, HID)),
            full((NT, HID, 64)),
            full((NT, 64)),
            full((NT, 64)),
            full((1, NT)),
        ],
        out_specs=pl.BlockSpec((EB, 1), lambda i: (i, 0)),
        out_shape=jax.ShapeDtypeStruct((NEDGES, 1), jnp.float32),
    )(ga, gb, attr, maskf, el, w1c, b1, w2, b2, w3, b3, w4, b4)


def _scatter_kernel(row_hbm, col_hbm, ty_hbm, at_hbm, q_hbm,
                    cx_hbm, cy_hbm, cz_hbm, z_hbm, out_hbm,
                    r_v, c_v, t_v, a_v, q_v, gxr, gyr, gzr, gxc, gyc, gzc,
                    bvx, bvy, bvz, bnx, bny, bnz, ba, bone,
                    sir_v, sic_v, sem, *accs):
    cid = lax.axis_index("c")
    sid = lax.axis_index("s")
    wid = sid * NC + cid
    base_w = wid * EPW

    pltpu.sync_copy(row_hbm.at[pl.ds(base_w, EPW)], r_v)
    pltpu.sync_copy(col_hbm.at[pl.ds(base_w, EPW)], c_v)
    pltpu.sync_copy(ty_hbm.at[pl.ds(base_w, EPW)], t_v)
    pltpu.sync_copy(at_hbm.at[pl.ds(base_w, EPW)], a_v)
    pltpu.sync_copy(q_hbm.at[pl.ds(base_w, EPW)], q_v)

    for k in range(NACC):
        @pl.when(sid == k)
        def _zero(k=k):
            pltpu.sync_copy(z_hbm, accs[k])

    ones = jnp.full((16,), 1.0, jnp.float32)
    for g in range(SCG):
        bone[pl.ds(g * 16, 16)] = ones
    plsc.subcore_barrier()

    def body(ch, carry):
        off = ch * SCC
        ri = r_v.at[pl.ds(off, SCC)]
        ci = c_v.at[pl.ds(off, SCC)]
        gs = [
            pltpu.async_copy(cx_hbm.at[ri], gxr, sem),
            pltpu.async_copy(cy_hbm.at[ri], gyr, sem),
            pltpu.async_copy(cz_hbm.at[ri], gzr, sem),
            pltpu.async_copy(cx_hbm.at[ci], gxc, sem),
            pltpu.async_copy(cy_hbm.at[ci], gyc, sem),
            pltpu.async_copy(cz_hbm.at[ci], gzc, sem),
        ]
        for d in gs:
            d.wait()
        for g in range(SCG):
            oe = off + g * 16
            r = r_v[pl.ds(oe, 16)]
            c = c_v[pl.ds(oe, 16)]
            tt = t_v[pl.ds(oe, 16)]
            av = a_v[pl.ds(oe, 16)]
            qv = q_v[pl.ds(oe, 16)]
            s16 = pl.ds(g * 16, 16)
            vx = qv * (gxr[s16] - gxc[s16])
            vy = qv * (gyr[s16] - gyc[s16])
            vz = qv * (gzr[s16] - gzc[s16])
            sir_v[s16] = r * NT + tt
            sic_v[s16] = c * NT + tt
            bvx[s16] = vx
            bvy[s16] = vy
            bvz[s16] = vz
            bnx[s16] = -vx
            bny[s16] = -vy
            bnz[s16] = -vz
            ba[s16] = av
        ds = [
            pltpu.async_copy(bvx, accs[0].at[sir_v], sem, add=True),
            pltpu.async_copy(bvy, accs[1].at[sir_v], sem, add=True),
            pltpu.async_copy(bvz, accs[2].at[sir_v], sem, add=True),
            pltpu.async_copy(ba, accs[3].at[sir_v], sem, add=True),
            pltpu.async_copy(bone, accs[4].at[sir_v], sem, add=True),
            pltpu.async_copy(bnx, accs[0].at[sic_v], sem, add=True),
            pltpu.async_copy(bny, accs[1].at[sic_v], sem, add=True),
            pltpu.async_copy(bnz, accs[2].at[sic_v], sem, add=True),
            pltpu.async_copy(ba, accs[5].at[sic_v], sem, add=True),
            pltpu.async_copy(bone, accs[6].at[sic_v], sem, add=True),
        ]
        for d in ds:
            d.wait()
        return carry

    lax.fori_loop(0, SCH, body, 0)
    plsc.subcore_barrier()

    for k in range(NACC):
        @pl.when(sid == k)
        def _writeback(k=k):
            pltpu.sync_copy(accs[k],
                            out_hbm.at[pl.ds((cid * NACC + k) * NROWSP, NROWSP)])


def _scatter(row, col, ty, attr, q, cx, cy, cz, zeros):
    mesh = plsc.VectorSubcoreMesh(core_axis_name="c", subcore_axis_name="s",
                                  num_cores=NC, num_subcores=NS)
    fn = functools.partial(
        pl.kernel,
        out_type=jax.ShapeDtypeStruct((NC * NACC * NROWSP,), jnp.float32),
        mesh=mesh,
        scratch_types=[
            pltpu.VMEM((EPW,), jnp.int32),
            pltpu.VMEM((EPW,), jnp.int32),
            pltpu.VMEM((EPW,), jnp.int32),
            pltpu.VMEM((EPW,), jnp.float32),
            pltpu.VMEM((EPW,), jnp.float32),
            pltpu.VMEM((SCC,), jnp.float32),
            pltpu.VMEM((SCC,), jnp.float32),
            pltpu.VMEM((SCC,), jnp.float32),
            pltpu.VMEM((SCC,), jnp.float32),
            pltpu.VMEM((SCC,), jnp.float32),
            pltpu.VMEM((SCC,), jnp.float32),
            pltpu.VMEM((SCC,), jnp.float32),
            pltpu.VMEM((SCC,), jnp.float32),
            pltpu.VMEM((SCC,), jnp.float32),
            pltpu.VMEM((SCC,), jnp.float32),
            pltpu.VMEM((SCC,), jnp.float32),
            pltpu.VMEM((SCC,), jnp.float32),
            pltpu.VMEM((SCC,), jnp.float32),
            pltpu.VMEM((SCC,), jnp.float32),
            pltpu.VMEM((SCC,), jnp.int32),
            pltpu.VMEM((SCC,), jnp.int32),
            pltpu.SemaphoreType.DMA,
        ] + [pltpu.VMEM_SHARED((NROWSP,), jnp.float32) for _ in range(NACC)],
    )(_scatter_kernel)
    return fn(row, col, ty, attr, q, cx, cy, cz, zeros)


def _final_body(acc_ref, h_ref, coord_ref, wg1h_ref, wg1e_ref, bg1_ref,
                wg2_ref, bg2_ref, out_ref):
    acc = acc_ref[0] + acc_ref[1]
    logits = []
    eqs = []
    for t in range(NT):
        e0 = acc[3, :, t:t + 1] / jnp.maximum(acc[4, :, t:t + 1], 1.0)
        e1 = acc[5, :, t:t + 1] / jnp.maximum(acc[6, :, t:t + 1], 1.0)
        z = (jnp.dot(h_ref[:, t, :], wg1h_ref[:], preferred_element_type=jnp.float32)
             + e0 * wg1e_ref[0:1, :] + e1 * wg1e_ref[1:2, :] + bg1_ref[:])
        x = _silu(z)
        lg = jnp.sum(x * wg2_ref[:], axis=1, keepdims=True) + bg2_ref[0, 0]
        logits.append(lg)
        eqs.append(jnp.concatenate(
            [acc[0, :, t:t + 1], acc[1, :, t:t + 1], acc[2, :, t:t + 1]], axis=1))
    lg = jnp.concatenate(logits, axis=1)
    m = jnp.max(lg, axis=1, keepdims=True)
    w = jnp.exp(lg - m)
    w = w / jnp.sum(w, axis=1, keepdims=True)
    out = coord_ref[:]
    for t in range(NT):
        out = out + w[:, t:t + 1] * eqs[t]
    out_ref[:] = out


def _final(acc4, h, coord, wg1h, wg1e, bg1, wg2, bg2):
    n = h.shape[0]
    grid = n // NB
    full = lambda shape: pl.BlockSpec(shape, lambda i: tuple(0 for _ in shape))
    return pl.pallas_call(
        _final_body,
        grid=(grid,),
        in_specs=[
            pl.BlockSpec((NC, NACC, NB, NT), lambda i: (0, 0, i, 0)),
            pl.BlockSpec((NB, NT, HID), lambda i: (i, 0, 0)),
            pl.BlockSpec((NB, 3), lambda i: (i, 0)),
            full((HID, HID)),
            full((2, HID)),
            full((1, HID)),
            full((1, HID)),
            full((1, 1)),
        ],
        out_specs=pl.BlockSpec((NB, 3), lambda i: (i, 0)),
        out_shape=jax.ShapeDtypeStruct((n, 3), jnp.float32),
    )(acc4, h, coord, wg1h, wg1e, bg1, wg2, bg2)


def kernel(h, coord, edge_index, coord_diff, edge_attr, edge_mask, edge_length,
           N, params):
    names = ["bond", "angle", "torsion", "radius"]
    row = edge_index[0]
    col = edge_index[1]
    etype = jnp.argmax(edge_mask, axis=0).astype(jnp.int32)
    gr = row * NT + etype
    gc = col * NT + etype
    maskf = edge_mask.T.astype(jnp.float32)

    def stack(i):
        ws = jnp.stack([params["mlp_" + n][i][0] for n in names])
        bs = jnp.stack([params["mlp_" + n][i][1] for n in names])
        return ws, bs

    w1, b1 = stack(0)
    wa = w1[:, :HID, :]
    wb = w1[:, HID:2 * HID, :]
    w1c = w1[:, 2 * HID, :]
    w2, b2 = stack(1)
    w3, b3 = stack(2)
    w2 = w2.astype(jnp.bfloat16)
    w3 = w3.astype(jnp.bfloat16)
    w4, b4 = stack(3)
    w4 = w4[:, :, 0]
    b4 = b4.reshape(1, NT)

    (wg1, bg1), (wg2, bg2) = params["w_gen"]
    wg1h = wg1[:HID, :]
    wg1e = wg1[HID:HID + 2, :]
    bg1 = bg1.reshape(1, HID)
    wg2 = wg2[:, 0].reshape(1, HID)
    bg2 = bg2.reshape(1, 1)

    a3, b3t = _proj(h, wa, wb)
    a2 = jax.lax.bitcast_convert_type(
        a3.reshape(NNODES * NT, D1 // 2, 2), jnp.int32)
    b2t = jax.lax.bitcast_convert_type(
        b3t.reshape(NNODES * NT, D1 // 2, 2), jnp.int32)

    ga_i, gb_i = _gather(a2, b2t, gr, gc)
    ga = jax.lax.bitcast_convert_type(ga_i, jnp.bfloat16).reshape(NEDGES, D1)
    gb = jax.lax.bitcast_convert_type(gb_i, jnp.bfloat16).reshape(NEDGES, D1)

    q = _edge_mlp(ga, gb, edge_attr, maskf, edge_length,
                  w1c, b1, w2, b2, w3, b3, w4, b4)

    zeros = jnp.zeros((NROWSP,), jnp.float32)
    acc = _scatter(row, col, etype, edge_attr[:, 0], q[:, 0],
                   coord[:, 0], coord[:, 1], coord[:, 2], zeros)
    acc4 = acc.reshape(NC, NACC, NROWSP)[:, :, :NROWS].reshape(
        NC, NACC, NNODES, NT)

    out = _final(acc4, h, coord, wg1h, wg1e, bg1, wg2, bg2)
    return out


# speedup vs baseline: 3.5261x; 3.5261x over previous
import functools

import jax
import jax.numpy as jnp
from jax import lax
from jax.experimental import pallas as pl
from jax.experimental.pallas import tpu as pltpu
from jax.experimental.pallas import tpu_sc as plsc

HID = 128
NT = 4
D1 = 2 * HID
NNODES = 10000
NEDGES = 320000

NC = 2
NS = 16
NW = NC * NS

EPW = NEDGES // NW
GK = 80
GCH = EPW // GK
SCC = 80
SCH = EPW // SCC
SCG = SCC // 16

NROWS = NNODES * NT
NROWSP = 40064
NACC = 7

EB = 1280
NB = 1000


def _silu(x):
    return x * (0.5 * jnp.tanh(0.5 * x) + 0.5)


def _proj_body(h_ref, wa_ref, wb_ref, a_ref, b_ref):
    for t in range(NT):
        ht = h_ref[:, t, :]
        a_ref[:, t, :] = jnp.dot(ht, wa_ref[t], preferred_element_type=jnp.float32)
        b_ref[:, t, :] = jnp.dot(ht, wb_ref[t], preferred_element_type=jnp.float32)


def _proj(h, wa, wb):
    n = h.shape[0]
    grid = n // NB
    return pl.pallas_call(
        _proj_body,
        grid=(grid,),
        in_specs=[
            pl.BlockSpec((NB, NT, HID), lambda i: (i, 0, 0)),
            pl.BlockSpec((NT, HID, D1), lambda i: (0, 0, 0)),
            pl.BlockSpec((NT, HID, D1), lambda i: (0, 0, 0)),
        ],
        out_specs=[
            pl.BlockSpec((NB, NT, D1), lambda i: (i, 0, 0)),
            pl.BlockSpec((NB, NT, D1), lambda i: (i, 0, 0)),
        ],
        out_shape=[
            jax.ShapeDtypeStruct((n, NT, D1), jnp.float32),
            jax.ShapeDtypeStruct((n, NT, D1), jnp.float32),
        ],
    )(h, wa, wb)


def _gather_kernel(a_hbm, b_hbm, gr_hbm, gc_hbm, ga_hbm, gb_hbm,
                   ir_v, ic_v, ra_v, rb_v, sa, sb):
    cid = lax.axis_index("c")
    sid = lax.axis_index("s")
    wid = sid * NC + cid
    base_w = wid * EPW

    def body(i, carry):
        base = base_w + i * GK
        pltpu.sync_copy(gr_hbm.at[pl.ds(base, GK)], ir_v)
        pltpu.sync_copy(gc_hbm.at[pl.ds(base, GK)], ic_v)
        da = pltpu.async_copy(a_hbm.at[ir_v], ra_v, sa)
        db = pltpu.async_copy(b_hbm.at[ic_v], rb_v, sb)
        da.wait()
        db.wait()
        pltpu.sync_copy(ra_v, ga_hbm.at[pl.ds(base, GK)])
        pltpu.sync_copy(rb_v, gb_hbm.at[pl.ds(base, GK)])
        return carry

    lax.fori_loop(0, GCH, body, 0)


def _gather(a2, b2, gr, gc):
    mesh = plsc.VectorSubcoreMesh(core_axis_name="c", subcore_axis_name="s",
                                  num_cores=NC, num_subcores=NS)
    fn = functools.partial(
        pl.kernel,
        out_type=(jax.ShapeDtypeStruct((NEDGES, D1), jnp.float32),
                  jax.ShapeDtypeStruct((NEDGES, D1), jnp.float32)),
        mesh=mesh,
        scratch_types=[
            pltpu.VMEM((GK,), jnp.int32),
            pltpu.VMEM((GK,), jnp.int32),
            pltpu.VMEM((GK, D1), jnp.float32),
            pltpu.VMEM((GK, D1), jnp.float32),
            pltpu.SemaphoreType.DMA,
            pltpu.SemaphoreType.DMA,
        ],
    )(_gather_kernel)
    return fn(a2, b2, gr, gc)


def _mlp_body(ga_ref, gb_ref, attr_ref, mask_ref, el_ref,
              w1c_ref, b1_ref, w2_ref, b2_ref, w3_ref, b3_ref, w4_ref, b4_ref,
              q_ref):
    g = ga_ref[:] + gb_ref[:]
    a = attr_ref[:]
    acc = jnp.zeros((g.shape[0], 1), jnp.float32)
    for t in range(NT):
        l1 = g + a * w1c_ref[t] + b1_ref[t]
        x = _silu(l1).astype(jnp.bfloat16)
        x = _silu(jnp.dot(x, w2_ref[t], preferred_element_type=jnp.float32)
                  + b2_ref[t]).astype(jnp.bfloat16)
        x = _silu(jnp.dot(x, w3_ref[t], preferred_element_type=jnp.float32)
                  + b3_ref[t])
        s = jnp.sum(x * w4_ref[t], axis=1, keepdims=True) + b4_ref[0, t]
        acc = acc + mask_ref[:, t:t + 1] * s
    q_ref[:] = acc / el_ref[:]


def _edge_mlp(ga, gb, attr, maskf, el, w1c, b1, w2, b2, w3, b3, w4, b4):
    grid = NEDGES // EB
    full = lambda shape: pl.BlockSpec(shape, lambda i: tuple(0 for _ in shape))
    return pl.pallas_call(
        _mlp_body,
        grid=(grid,),
        in_specs=[
            pl.BlockSpec((EB, D1), lambda i: (i, 0)),
            pl.BlockSpec((EB, D1), lambda i: (i, 0)),
            pl.BlockSpec((EB, 1), lambda i: (i, 0)),
            pl.BlockSpec((EB, NT), lambda i: (i, 0)),
            pl.BlockSpec((EB, 1), lambda i: (i, 0)),
            full((NT, D1)),
            full((NT, D1)),
            full((NT, D1, HID)),
            full((NT, HID)),
            full((NT, HID, 64)),
            full((NT, 64)),
            full((NT, 64)),
            full((1, NT)),
        ],
        out_specs=pl.BlockSpec((EB, 1), lambda i: (i, 0)),
        out_shape=jax.ShapeDtypeStruct((NEDGES, 1), jnp.float32),
    )(ga, gb, attr, maskf, el, w1c, b1, w2, b2, w3, b3, w4, b4)


def _scatter_kernel(row_hbm, col_hbm, ty_hbm, at_hbm, q_hbm,
                    cx_hbm, cy_hbm, cz_hbm, z_hbm, out_hbm,
                    r_v, c_v, t_v, a_v, q_v, gxr, gyr, gzr, gxc, gyc, gzc,
                    bvx, bvy, bvz, bnx, bny, bnz, ba, bone,
                    sir_v, sic_v, sem, *accs):
    cid = lax.axis_index("c")
    sid = lax.axis_index("s")
    wid = sid * NC + cid
    base_w = wid * EPW

    pltpu.sync_copy(row_hbm.at[pl.ds(base_w, EPW)], r_v)
    pltpu.sync_copy(col_hbm.at[pl.ds(base_w, EPW)], c_v)
    pltpu.sync_copy(ty_hbm.at[pl.ds(base_w, EPW)], t_v)
    pltpu.sync_copy(at_hbm.at[pl.ds(base_w, EPW)], a_v)
    pltpu.sync_copy(q_hbm.at[pl.ds(base_w, EPW)], q_v)

    for k in range(NACC):
        @pl.when(sid == k)
        def _zero(k=k):
            pltpu.sync_copy(z_hbm, accs[k])

    ones = jnp.full((16,), 1.0, jnp.float32)
    for g in range(SCG):
        bone[pl.ds(g * 16, 16)] = ones
    plsc.subcore_barrier()

    def body(ch, carry):
        off = ch * SCC
        ri = r_v.at[pl.ds(off, SCC)]
        ci = c_v.at[pl.ds(off, SCC)]
        gs = [
            pltpu.async_copy(cx_hbm.at[ri], gxr, sem),
            pltpu.async_copy(cy_hbm.at[ri], gyr, sem),
            pltpu.async_copy(cz_hbm.at[ri], gzr, sem),
            pltpu.async_copy(cx_hbm.at[ci], gxc, sem),
            pltpu.async_copy(cy_hbm.at[ci], gyc, sem),
            pltpu.async_copy(cz_hbm.at[ci], gzc, sem),
        ]
        for d in gs:
            d.wait()
        for g in range(SCG):
            oe = off + g * 16
            r = r_v[pl.ds(oe, 16)]
            c = c_v[pl.ds(oe, 16)]
            tt = t_v[pl.ds(oe, 16)]
            av = a_v[pl.ds(oe, 16)]
            qv = q_v[pl.ds(oe, 16)]
            s16 = pl.ds(g * 16, 16)
            vx = qv * (gxr[s16] - gxc[s16])
            vy = qv * (gyr[s16] - gyc[s16])
            vz = qv * (gzr[s16] - gzc[s16])
            sir_v[s16] = r * NT + tt
            sic_v[s16] = c * NT + tt
            bvx[s16] = vx
            bvy[s16] = vy
            bvz[s16] = vz
            bnx[s16] = -vx
            bny[s16] = -vy
            bnz[s16] = -vz
            ba[s16] = av
        ds = [
            pltpu.async_copy(bvx, accs[0].at[sir_v], sem, add=True),
            pltpu.async_copy(bvy, accs[1].at[sir_v], sem, add=True),
            pltpu.async_copy(bvz, accs[2].at[sir_v], sem, add=True),
            pltpu.async_copy(ba, accs[3].at[sir_v], sem, add=True),
            pltpu.async_copy(bone, accs[4].at[sir_v], sem, add=True),
            pltpu.async_copy(bnx, accs[0].at[sic_v], sem, add=True),
            pltpu.async_copy(bny, accs[1].at[sic_v], sem, add=True),
            pltpu.async_copy(bnz, accs[2].at[sic_v], sem, add=True),
            pltpu.async_copy(ba, accs[5].at[sic_v], sem, add=True),
            pltpu.async_copy(bone, accs[6].at[sic_v], sem, add=True),
        ]
        for d in ds:
            d.wait()
        return carry

    lax.fori_loop(0, SCH, body, 0)
    plsc.subcore_barrier()

    for k in range(NACC):
        @pl.when(sid == k)
        def _writeback(k=k):
            pltpu.sync_copy(accs[k],
                            out_hbm.at[pl.ds((cid * NACC + k) * NROWSP, NROWSP)])


def _scatter(row, col, ty, attr, q, cx, cy, cz, zeros):
    mesh = plsc.VectorSubcoreMesh(core_axis_name="c", subcore_axis_name="s",
                                  num_cores=NC, num_subcores=NS)
    fn = functools.partial(
        pl.kernel,
        out_type=jax.ShapeDtypeStruct((NC * NACC * NROWSP,), jnp.float32),
        mesh=mesh,
        scratch_types=[
            pltpu.VMEM((EPW,), jnp.int32),
            pltpu.VMEM((EPW,), jnp.int32),
            pltpu.VMEM((EPW,), jnp.int32),
            pltpu.VMEM((EPW,), jnp.float32),
            pltpu.VMEM((EPW,), jnp.float32),
            pltpu.VMEM((SCC,), jnp.float32),
            pltpu.VMEM((SCC,), jnp.float32),
            pltpu.VMEM((SCC,), jnp.float32),
            pltpu.VMEM((SCC,), jnp.float32),
            pltpu.VMEM((SCC,), jnp.float32),
            pltpu.VMEM((SCC,), jnp.float32),
            pltpu.VMEM((SCC,), jnp.float32),
            pltpu.VMEM((SCC,), jnp.float32),
            pltpu.VMEM((SCC,), jnp.float32),
            pltpu.VMEM((SCC,), jnp.float32),
            pltpu.VMEM((SCC,), jnp.float32),
            pltpu.VMEM((SCC,), jnp.float32),
            pltpu.VMEM((SCC,), jnp.float32),
            pltpu.VMEM((SCC,), jnp.float32),
            pltpu.VMEM((SCC,), jnp.int32),
            pltpu.VMEM((SCC,), jnp.int32),
            pltpu.SemaphoreType.DMA,
        ] + [pltpu.VMEM_SHARED((NROWSP,), jnp.float32) for _ in range(NACC)],
    )(_scatter_kernel)
    return fn(row, col, ty, attr, q, cx, cy, cz, zeros)


def _final_body(acc_ref, h_ref, coord_ref, wg1h_ref, wg1e_ref, bg1_ref,
                wg2_ref, bg2_ref, out_ref):
    acc = acc_ref[0] + acc_ref[1]
    logits = []
    eqs = []
    for t in range(NT):
        e0 = acc[3, :, t:t + 1] / jnp.maximum(acc[4, :, t:t + 1], 1.0)
        e1 = acc[5, :, t:t + 1] / jnp.maximum(acc[6, :, t:t + 1], 1.0)
        z = (jnp.dot(h_ref[:, t, :], wg1h_ref[:], preferred_element_type=jnp.float32)
             + e0 * wg1e_ref[0:1, :] + e1 * wg1e_ref[1:2, :] + bg1_ref[:])
        x = _silu(z)
        lg = jnp.sum(x * wg2_ref[:], axis=1, keepdims=True) + bg2_ref[0, 0]
        logits.append(lg)
        eqs.append(jnp.concatenate(
            [acc[0, :, t:t + 1], acc[1, :, t:t + 1], acc[2, :, t:t + 1]], axis=1))
    lg = jnp.concatenate(logits, axis=1)
    m = jnp.max(lg, axis=1, keepdims=True)
    w = jnp.exp(lg - m)
    w = w / jnp.sum(w, axis=1, keepdims=True)
    out = coord_ref[:]
    for t in range(NT):
        out = out + w[:, t:t + 1] * eqs[t]
    out_ref[:] = out


def _final(acc4, h, coord, wg1h, wg1e, bg1, wg2, bg2):
    n = h.shape[0]
    grid = n // NB
    full = lambda shape: pl.BlockSpec(shape, lambda i: tuple(0 for _ in shape))
    return pl.pallas_call(
        _final_body,
        grid=(grid,),
        in_specs=[
            pl.BlockSpec((NC, NACC, NB, NT), lambda i: (0, 0, i, 0)),
            pl.BlockSpec((NB, NT, HID), lambda i: (i, 0, 0)),
            pl.BlockSpec((NB, 3), lambda i: (i, 0)),
            full((HID, HID)),
            full((2, HID)),
            full((1, HID)),
            full((1, HID)),
            full((1, 1)),
        ],
        out_specs=pl.BlockSpec((NB, 3), lambda i: (i, 0)),
        out_shape=jax.ShapeDtypeStruct((n, 3), jnp.float32),
    )(acc4, h, coord, wg1h, wg1e, bg1, wg2, bg2)


def kernel(h, coord, edge_index, coord_diff, edge_attr, edge_mask, edge_length,
           N, params):
    names = ["bond", "angle", "torsion", "radius"]
    row = edge_index[0]
    col = edge_index[1]
    etype = jnp.argmax(edge_mask, axis=0).astype(jnp.int32)
    gr = row * NT + etype
    gc = col * NT + etype
    maskf = edge_mask.T.astype(jnp.float32)

    def stack(i):
        ws = jnp.stack([params["mlp_" + n][i][0] for n in names])
        bs = jnp.stack([params["mlp_" + n][i][1] for n in names])
        return ws, bs

    w1, b1 = stack(0)
    wa = w1[:, :HID, :]
    wb = w1[:, HID:2 * HID, :]
    w1c = w1[:, 2 * HID, :]
    w2, b2 = stack(1)
    w3, b3 = stack(2)
    w2 = w2.astype(jnp.bfloat16)
    w3 = w3.astype(jnp.bfloat16)
    w4, b4 = stack(3)
    w4 = w4[:, :, 0]
    b4 = b4.reshape(1, NT)

    (wg1, bg1), (wg2, bg2) = params["w_gen"]
    wg1h = wg1[:HID, :]
    wg1e = wg1[HID:HID + 2, :]
    bg1 = bg1.reshape(1, HID)
    wg2 = wg2[:, 0].reshape(1, HID)
    bg2 = bg2.reshape(1, 1)

    a3, b3t = _proj(h, wa, wb)
    a2 = a3.reshape(NNODES * NT, D1)
    b2t = b3t.reshape(NNODES * NT, D1)

    ga, gb = _gather(a2, b2t, gr, gc)

    q = _edge_mlp(ga, gb, edge_attr, maskf, edge_length,
                  w1c, b1, w2, b2, w3, b3, w4, b4)

    zeros = jnp.zeros((NROWSP,), jnp.float32)
    acc = _scatter(row, col, etype, edge_attr[:, 0], q[:, 0],
                   coord[:, 0], coord[:, 1], coord[:, 2], zeros)
    acc4 = acc.reshape(NC, NACC, NROWSP)[:, :, :NROWS].reshape(
        NC, NACC, NNODES, NT)

    out = _final(acc4, h, coord, wg1h, wg1e, bg1, wg2, bg2)
    return out


# speedup vs baseline: 3.7965x; 1.0767x over previous
import functools

import jax
import jax.numpy as jnp
from jax import lax
from jax.experimental import pallas as pl
from jax.experimental.pallas import tpu as pltpu
from jax.experimental.pallas import tpu_sc as plsc

HID = 128
NT = 4
D1 = 2 * HID
NNODES = 10000
NEDGES = 320000

NC = 2
NS = 16
NW = NC * NS

EPW = NEDGES // NW
GK = 80
GCH = EPW // GK
SCC = 80
SCH = EPW // SCC
SCG = SCC // 16

NROWS = NNODES * NT
NROWSP = 40064
NACC = 7

EB = 1280
NB = 1000


def _silu(x):
    return x * (0.5 * jnp.tanh(0.5 * x) + 0.5)


def _proj_body(h_ref, wa_ref, wb_ref, a_ref, b_ref):
    for t in range(NT):
        ht = h_ref[:, t, :]
        a_ref[:, t, :] = jnp.dot(ht, wa_ref[t], preferred_element_type=jnp.float32)
        b_ref[:, t, :] = jnp.dot(ht, wb_ref[t], preferred_element_type=jnp.float32)


def _proj(h, wa, wb):
    n = h.shape[0]
    grid = n // NB
    return pl.pallas_call(
        _proj_body,
        grid=(grid,),
        in_specs=[
            pl.BlockSpec((NB, NT, HID), lambda i: (i, 0, 0)),
            pl.BlockSpec((NT, HID, D1), lambda i: (0, 0, 0)),
            pl.BlockSpec((NT, HID, D1), lambda i: (0, 0, 0)),
        ],
        out_specs=[
            pl.BlockSpec((NB, NT, D1), lambda i: (i, 0, 0)),
            pl.BlockSpec((NB, NT, D1), lambda i: (i, 0, 0)),
        ],
        out_shape=[
            jax.ShapeDtypeStruct((n, NT, D1), jnp.float32),
            jax.ShapeDtypeStruct((n, NT, D1), jnp.float32),
        ],
    )(h, wa, wb)


def _gather_kernel(a_hbm, b_hbm, gr_hbm, gc_hbm, ga_hbm, gb_hbm,
                   ir_v, ic_v, ra_v, rb_v, sa, sb):
    cid = lax.axis_index("c")
    sid = lax.axis_index("s")
    wid = sid * NC + cid
    base_w = wid * EPW

    def body(i, carry):
        base = base_w + i * GK
        pltpu.sync_copy(gr_hbm.at[pl.ds(base, GK)], ir_v)
        pltpu.sync_copy(gc_hbm.at[pl.ds(base, GK)], ic_v)
        da = pltpu.async_copy(a_hbm.at[ir_v], ra_v, sa)
        db = pltpu.async_copy(b_hbm.at[ic_v], rb_v, sb)
        da.wait()
        db.wait()
        pltpu.sync_copy(ra_v, ga_hbm.at[pl.ds(base, GK)])
        pltpu.sync_copy(rb_v, gb_hbm.at[pl.ds(base, GK)])
        return carry

    lax.fori_loop(0, GCH, body, 0)


def _gather(a2, b2, gr, gc):
    mesh = plsc.VectorSubcoreMesh(core_axis_name="c", subcore_axis_name="s",
                                  num_cores=NC, num_subcores=NS)
    fn = functools.partial(
        pl.kernel,
        out_type=(jax.ShapeDtypeStruct((NEDGES, D1), jnp.float32),
                  jax.ShapeDtypeStruct((NEDGES, D1), jnp.float32)),
        mesh=mesh,
        scratch_types=[
            pltpu.VMEM((GK,), jnp.int32),
            pltpu.VMEM((GK,), jnp.int32),
            pltpu.VMEM((GK, D1), jnp.float32),
            pltpu.VMEM((GK, D1), jnp.float32),
            pltpu.SemaphoreType.DMA,
            pltpu.SemaphoreType.DMA,
        ],
    )(_gather_kernel)
    return fn(a2, b2, gr, gc)


def _mlp_body(ga_ref, gb_ref, attr_ref, mask_ref, el_ref,
              w1c_ref, b1_ref, w2_ref, b2_ref, w3_ref, b3_ref, w4_ref, b4_ref,
              q_ref):
    g = ga_ref[:] + gb_ref[:]
    a = attr_ref[:]
    m = mask_ref[:]
    w1cs = jnp.dot(m, w1c_ref[:], preferred_element_type=jnp.float32)
    b1s = jnp.dot(m, b1_ref[:], preferred_element_type=jnp.float32)
    x1 = _silu(g + a * w1cs + b1s).astype(jnp.bfloat16)
    acc = jnp.zeros((g.shape[0], 1), jnp.float32)
    for t in range(NT):
        x = _silu(jnp.dot(x1, w2_ref[t], preferred_element_type=jnp.float32)
                  + b2_ref[t]).astype(jnp.bfloat16)
        x = _silu(jnp.dot(x, w3_ref[t], preferred_element_type=jnp.float32)
                  + b3_ref[t])
        s = jnp.sum(x * w4_ref[t], axis=1, keepdims=True) + b4_ref[0, t]
        acc = acc + m[:, t:t + 1] * s
    q_ref[:] = acc / el_ref[:]


def _edge_mlp(ga, gb, attr, maskf, el, w1c, b1, w2, b2, w3, b3, w4, b4):
    grid = NEDGES // EB
    full = lambda shape: pl.BlockSpec(shape, lambda i: tuple(0 for _ in shape))
    return pl.pallas_call(
        _mlp_body,
        grid=(grid,),
        in_specs=[
            pl.BlockSpec((EB, D1), lambda i: (i, 0)),
            pl.BlockSpec((EB, D1), lambda i: (i, 0)),
            pl.BlockSpec((EB, 1), lambda i: (i, 0)),
            pl.BlockSpec((EB, NT), lambda i: (i, 0)),
            pl.BlockSpec((EB, 1), lambda i: (i, 0)),
            full((NT, D1)),
            full((NT, D1)),
            full((NT, D1, HID)),
            full((NT, HID)),
            full((NT, HID, 64)),
            full((NT, 64)),
            full((NT, 64)),
            full((1, NT)),
        ],
        out_specs=pl.BlockSpec((EB, 1), lambda i: (i, 0)),
        out_shape=jax.ShapeDtypeStruct((NEDGES, 1), jnp.float32),
    )(ga, gb, attr, maskf, el, w1c, b1, w2, b2, w3, b3, w4, b4)


def _scatter_kernel(row_hbm, col_hbm, ty_hbm, at_hbm, q_hbm,
                    cx_hbm, cy_hbm, cz_hbm, z_hbm, out_hbm,
                    r_v, c_v, t_v, a_v, q_v, gxr, gyr, gzr, gxc, gyc, gzc,
                    bvx, bvy, bvz, bnx, bny, bnz, ba, bone,
                    sir_v, sic_v, sem, *accs):
    cid = lax.axis_index("c")
    sid = lax.axis_index("s")
    wid = sid * NC + cid
    base_w = wid * EPW

    pltpu.sync_copy(row_hbm.at[pl.ds(base_w, EPW)], r_v)
    pltpu.sync_copy(col_hbm.at[pl.ds(base_w, EPW)], c_v)
    pltpu.sync_copy(ty_hbm.at[pl.ds(base_w, EPW)], t_v)
    pltpu.sync_copy(at_hbm.at[pl.ds(base_w, EPW)], a_v)
    pltpu.sync_copy(q_hbm.at[pl.ds(base_w, EPW)], q_v)

    for k in range(NACC):
        @pl.when(sid == k)
        def _zero(k=k):
            pltpu.sync_copy(z_hbm, accs[k])

    ones = jnp.full((16,), 1.0, jnp.float32)
    for g in range(SCG):
        bone[pl.ds(g * 16, 16)] = ones
    plsc.subcore_barrier()

    def body(ch, carry):
        off = ch * SCC
        ri = r_v.at[pl.ds(off, SCC)]
        ci = c_v.at[pl.ds(off, SCC)]
        gs = [
            pltpu.async_copy(cx_hbm.at[ri], gxr, sem),
            pltpu.async_copy(cy_hbm.at[ri], gyr, sem),
            pltpu.async_copy(cz_hbm.at[ri], gzr, sem),
            pltpu.async_copy(cx_hbm.at[ci], gxc, sem),
            pltpu.async_copy(cy_hbm.at[ci], gyc, sem),
            pltpu.async_copy(cz_hbm.at[ci], gzc, sem),
        ]
        for d in gs:
            d.wait()
        for g in range(SCG):
            oe = off + g * 16
            r = r_v[pl.ds(oe, 16)]
            c = c_v[pl.ds(oe, 16)]
            tt = t_v[pl.ds(oe, 16)]
            av = a_v[pl.ds(oe, 16)]
            qv = q_v[pl.ds(oe, 16)]
            s16 = pl.ds(g * 16, 16)
            vx = qv * (gxr[s16] - gxc[s16])
            vy = qv * (gyr[s16] - gyc[s16])
            vz = qv * (gzr[s16] - gzc[s16])
            sir_v[s16] = r * NT + tt
            sic_v[s16] = c * NT + tt
            bvx[s16] = vx
            bvy[s16] = vy
            bvz[s16] = vz
            bnx[s16] = -vx
            bny[s16] = -vy
            bnz[s16] = -vz
            ba[s16] = av
        ds = [
            pltpu.async_copy(bvx, accs[0].at[sir_v], sem, add=True),
            pltpu.async_copy(bvy, accs[1].at[sir_v], sem, add=True),
            pltpu.async_copy(bvz, accs[2].at[sir_v], sem, add=True),
            pltpu.async_copy(ba, accs[3].at[sir_v], sem, add=True),
            pltpu.async_copy(bone, accs[4].at[sir_v], sem, add=True),
            pltpu.async_copy(bnx, accs[0].at[sic_v], sem, add=True),
            pltpu.async_copy(bny, accs[1].at[sic_v], sem, add=True),
            pltpu.async_copy(bnz, accs[2].at[sic_v], sem, add=True),
            pltpu.async_copy(ba, accs[5].at[sic_v], sem, add=True),
            pltpu.async_copy(bone, accs[6].at[sic_v], sem, add=True),
        ]
        for d in ds:
            d.wait()
        return carry

    lax.fori_loop(0, SCH, body, 0)
    plsc.subcore_barrier()

    for k in range(NACC):
        @pl.when(sid == k)
        def _writeback(k=k):
            pltpu.sync_copy(accs[k],
                            out_hbm.at[pl.ds((cid * NACC + k) * NROWSP, NROWSP)])


def _scatter(row, col, ty, attr, q, cx, cy, cz, zeros):
    mesh = plsc.VectorSubcoreMesh(core_axis_name="c", subcore_axis_name="s",
                                  num_cores=NC, num_subcores=NS)
    fn = functools.partial(
        pl.kernel,
        out_type=jax.ShapeDtypeStruct((NC * NACC * NROWSP,), jnp.float32),
        mesh=mesh,
        scratch_types=[
            pltpu.VMEM((EPW,), jnp.int32),
            pltpu.VMEM((EPW,), jnp.int32),
            pltpu.VMEM((EPW,), jnp.int32),
            pltpu.VMEM((EPW,), jnp.float32),
            pltpu.VMEM((EPW,), jnp.float32),
            pltpu.VMEM((SCC,), jnp.float32),
            pltpu.VMEM((SCC,), jnp.float32),
            pltpu.VMEM((SCC,), jnp.float32),
            pltpu.VMEM((SCC,), jnp.float32),
            pltpu.VMEM((SCC,), jnp.float32),
            pltpu.VMEM((SCC,), jnp.float32),
            pltpu.VMEM((SCC,), jnp.float32),
            pltpu.VMEM((SCC,), jnp.float32),
            pltpu.VMEM((SCC,), jnp.float32),
            pltpu.VMEM((SCC,), jnp.float32),
            pltpu.VMEM((SCC,), jnp.float32),
            pltpu.VMEM((SCC,), jnp.float32),
            pltpu.VMEM((SCC,), jnp.float32),
            pltpu.VMEM((SCC,), jnp.float32),
            pltpu.VMEM((SCC,), jnp.int32),
            pltpu.VMEM((SCC,), jnp.int32),
            pltpu.SemaphoreType.DMA,
        ] + [pltpu.VMEM_SHARED((NROWSP,), jnp.float32) for _ in range(NACC)],
    )(_scatter_kernel)
    return fn(row, col, ty, attr, q, cx, cy, cz, zeros)


def _final_body(acc_ref, h_ref, coord_ref, wg1h_ref, wg1e_ref, bg1_ref,
                wg2_ref, bg2_ref, out_ref):
    acc = acc_ref[0] + acc_ref[1]
    logits = []
    eqs = []
    for t in range(NT):
        e0 = acc[3, :, t:t + 1] / jnp.maximum(acc[4, :, t:t + 1], 1.0)
        e1 = acc[5, :, t:t + 1] / jnp.maximum(acc[6, :, t:t + 1], 1.0)
        z = (jnp.dot(h_ref[:, t, :], wg1h_ref[:], preferred_element_type=jnp.float32)
             + e0 * wg1e_ref[0:1, :] + e1 * wg1e_ref[1:2, :] + bg1_ref[:])
        x = _silu(z)
        lg = jnp.sum(x * wg2_ref[:], axis=1, keepdims=True) + bg2_ref[0, 0]
        logits.append(lg)
        eqs.append(jnp.concatenate(
            [acc[0, :, t:t + 1], acc[1, :, t:t + 1], acc[2, :, t:t + 1]], axis=1))
    lg = jnp.concatenate(logits, axis=1)
    m = jnp.max(lg, axis=1, keepdims=True)
    w = jnp.exp(lg - m)
    w = w / jnp.sum(w, axis=1, keepdims=True)
    out = coord_ref[:]
    for t in range(NT):
        out = out + w[:, t:t + 1] * eqs[t]
    out_ref[:] = out


def _final(acc4, h, coord, wg1h, wg1e, bg1, wg2, bg2):
    n = h.shape[0]
    grid = n // NB
    full = lambda shape: pl.BlockSpec(shape, lambda i: tuple(0 for _ in shape))
    return pl.pallas_call(
        _final_body,
        grid=(grid,),
        in_specs=[
            pl.BlockSpec((NC, NACC, NB, NT), lambda i: (0, 0, i, 0)),
            pl.BlockSpec((NB, NT, HID), lambda i: (i, 0, 0)),
            pl.BlockSpec((NB, 3), lambda i: (i, 0)),
            full((HID, HID)),
            full((2, HID)),
            full((1, HID)),
            full((1, HID)),
            full((1, 1)),
        ],
        out_specs=pl.BlockSpec((NB, 3), lambda i: (i, 0)),
        out_shape=jax.ShapeDtypeStruct((n, 3), jnp.float32),
    )(acc4, h, coord, wg1h, wg1e, bg1, wg2, bg2)


def kernel(h, coord, edge_index, coord_diff, edge_attr, edge_mask, edge_length,
           N, params):
    names = ["bond", "angle", "torsion", "radius"]
    row = edge_index[0]
    col = edge_index[1]
    etype = jnp.argmax(edge_mask, axis=0).astype(jnp.int32)
    gr = row * NT + etype
    gc = col * NT + etype
    maskf = edge_mask.T.astype(jnp.float32)

    def stack(i):
        ws = jnp.stack([params["mlp_" + n][i][0] for n in names])
        bs = jnp.stack([params["mlp_" + n][i][1] for n in names])
        return ws, bs

    w1, b1 = stack(0)
    wa = w1[:, :HID, :]
    wb = w1[:, HID:2 * HID, :]
    w1c = w1[:, 2 * HID, :]
    w2, b2 = stack(1)
    w3, b3 = stack(2)
    w2 = w2.astype(jnp.bfloat16)
    w3 = w3.astype(jnp.bfloat16)
    w4, b4 = stack(3)
    w4 = w4[:, :, 0]
    b4 = b4.reshape(1, NT)

    (wg1, bg1), (wg2, bg2) = params["w_gen"]
    wg1h = wg1[:HID, :]
    wg1e = wg1[HID:HID + 2, :]
    bg1 = bg1.reshape(1, HID)
    wg2 = wg2[:, 0].reshape(1, HID)
    bg2 = bg2.reshape(1, 1)

    a3, b3t = _proj(h, wa, wb)
    a2 = a3.reshape(NNODES * NT, D1)
    b2t = b3t.reshape(NNODES * NT, D1)

    ga, gb = _gather(a2, b2t, gr, gc)

    q = _edge_mlp(ga, gb, edge_attr, maskf, edge_length,
                  w1c, b1, w2, b2, w3, b3, w4, b4)

    zeros = jnp.zeros((NROWSP,), jnp.float32)
    acc = _scatter(row, col, etype, edge_attr[:, 0], q[:, 0],
                   coord[:, 0], coord[:, 1], coord[:, 2], zeros)
    acc4 = acc.reshape(NC, NACC, NROWSP)[:, :, :NROWS].reshape(
        NC, NACC, NNODES, NT)

    out = _final(acc4, h, coord, wg1h, wg1e, bg1, wg2, bg2)
    return out


# speedup vs baseline: 4.9243x; 1.2971x over previous
import functools

import jax
import jax.numpy as jnp
from jax import lax
from jax.experimental import pallas as pl
from jax.experimental.pallas import tpu as pltpu
from jax.experimental.pallas import tpu_sc as plsc

HID = 128
NT = 4
D1 = 2 * HID
NNODES = 10000
NEDGES = 320000

NC = 2
NS = 16
NW = NC * NS

EPW = NEDGES // NW
GK = 80
GCH = EPW // GK
SCC = 80
SCH = EPW // SCC
SCG = SCC // 16

NROWS = NNODES * NT
NROWSP = 40064
NACC = 7

EB = 1280
NB = 1000


def _silu(x):
    return x * (0.5 * jnp.tanh(0.5 * x) + 0.5)


def _proj_body(h_ref, wa_ref, wb_ref, a_ref, b_ref):
    for t in range(NT):
        ht = h_ref[:, t, :]
        a_ref[:, t, :] = jnp.dot(ht, wa_ref[t], preferred_element_type=jnp.float32)
        b_ref[:, t, :] = jnp.dot(ht, wb_ref[t], preferred_element_type=jnp.float32)


def _proj(h, wa, wb):
    n = h.shape[0]
    grid = n // NB
    return pl.pallas_call(
        _proj_body,
        grid=(grid,),
        in_specs=[
            pl.BlockSpec((NB, NT, HID), lambda i: (i, 0, 0)),
            pl.BlockSpec((NT, HID, D1), lambda i: (0, 0, 0)),
            pl.BlockSpec((NT, HID, D1), lambda i: (0, 0, 0)),
        ],
        out_specs=[
            pl.BlockSpec((NB, NT, D1), lambda i: (i, 0, 0)),
            pl.BlockSpec((NB, NT, D1), lambda i: (i, 0, 0)),
        ],
        out_shape=[
            jax.ShapeDtypeStruct((n, NT, D1), jnp.float32),
            jax.ShapeDtypeStruct((n, NT, D1), jnp.float32),
        ],
    )(h, wa, wb)


def _gather_kernel(a_hbm, b_hbm, gr_hbm, gc_hbm, ga_hbm, gb_hbm,
                   ir_v, ic_v, ra_v, rb_v, sa, sb):
    cid = lax.axis_index("c")
    sid = lax.axis_index("s")
    wid = sid * NC + cid
    base_w = wid * EPW

    def body(i, carry):
        base = base_w + i * GK
        pltpu.sync_copy(gr_hbm.at[pl.ds(base, GK)], ir_v)
        pltpu.sync_copy(gc_hbm.at[pl.ds(base, GK)], ic_v)
        da = pltpu.async_copy(a_hbm.at[ir_v], ra_v, sa)
        db = pltpu.async_copy(b_hbm.at[ic_v], rb_v, sb)
        da.wait()
        db.wait()
        pltpu.sync_copy(ra_v, ga_hbm.at[pl.ds(base, GK)])
        pltpu.sync_copy(rb_v, gb_hbm.at[pl.ds(base, GK)])
        return carry

    lax.fori_loop(0, GCH, body, 0)


def _gather(a2, b2, gr, gc):
    mesh = plsc.VectorSubcoreMesh(core_axis_name="c", subcore_axis_name="s",
                                  num_cores=NC, num_subcores=NS)
    fn = functools.partial(
        pl.kernel,
        out_type=(jax.ShapeDtypeStruct((NEDGES, D1), jnp.float32),
                  jax.ShapeDtypeStruct((NEDGES, D1), jnp.float32)),
        mesh=mesh,
        scratch_types=[
            pltpu.VMEM((GK,), jnp.int32),
            pltpu.VMEM((GK,), jnp.int32),
            pltpu.VMEM((GK, D1), jnp.float32),
            pltpu.VMEM((GK, D1), jnp.float32),
            pltpu.SemaphoreType.DMA,
            pltpu.SemaphoreType.DMA,
        ],
    )(_gather_kernel)
    return fn(a2, b2, gr, gc)


def _mlp_body(ga_ref, gb_ref, attr_ref, mask_ref, el_ref,
              w1c_ref, b1_ref, w2_ref, b2_ref, w3_ref, b3_ref, w4_ref, b4_ref,
              q_ref):
    g = ga_ref[:] + gb_ref[:]
    a = attr_ref[:]
    m = mask_ref[:]
    w1cs = jnp.dot(m, w1c_ref[:], preferred_element_type=jnp.float32)
    b1s = jnp.dot(m, b1_ref[:], preferred_element_type=jnp.float32)
    x1 = _silu(g + a * w1cs + b1s).astype(jnp.bfloat16)
    l2 = jnp.dot(m, b2_ref[:], preferred_element_type=jnp.float32)
    for t in range(NT):
        l2 = l2 + m[:, t:t + 1] * jnp.dot(
            x1, w2_ref[t], preferred_element_type=jnp.float32)
    x2 = _silu(l2).astype(jnp.bfloat16)
    l3 = jnp.dot(m, b3_ref[:], preferred_element_type=jnp.float32)
    for t in range(NT):
        l3 = l3 + m[:, t:t + 1] * jnp.dot(
            x2, w3_ref[t], preferred_element_type=jnp.float32)
    x3 = _silu(l3)
    w4s = jnp.dot(m, w4_ref[:], preferred_element_type=jnp.float32)
    b4s = jnp.dot(m, b4_ref[:], preferred_element_type=jnp.float32)
    s = jnp.sum(x3 * w4s, axis=1, keepdims=True) + b4s
    q_ref[:] = s / el_ref[:]


def _edge_mlp(ga, gb, attr, maskf, el, w1c, b1, w2, b2, w3, b3, w4, b4):
    grid = NEDGES // EB
    full = lambda shape: pl.BlockSpec(shape, lambda i: tuple(0 for _ in shape))
    return pl.pallas_call(
        _mlp_body,
        grid=(grid,),
        in_specs=[
            pl.BlockSpec((EB, D1), lambda i: (i, 0)),
            pl.BlockSpec((EB, D1), lambda i: (i, 0)),
            pl.BlockSpec((EB, 1), lambda i: (i, 0)),
            pl.BlockSpec((EB, NT), lambda i: (i, 0)),
            pl.BlockSpec((EB, 1), lambda i: (i, 0)),
            full((NT, D1)),
            full((NT, D1)),
            full((NT, D1, HID)),
            full((NT, HID)),
            full((NT, HID, 64)),
            full((NT, 64)),
            full((NT, 64)),
            full((NT, 1)),
        ],
        out_specs=pl.BlockSpec((EB, 1), lambda i: (i, 0)),
        out_shape=jax.ShapeDtypeStruct((NEDGES, 1), jnp.float32),
    )(ga, gb, attr, maskf, el, w1c, b1, w2, b2, w3, b3, w4, b4)


def _scatter_kernel(row_hbm, col_hbm, ty_hbm, at_hbm, q_hbm,
                    cx_hbm, cy_hbm, cz_hbm, z_hbm, out_hbm,
                    r_v, c_v, t_v, a_v, q_v, gxr, gyr, gzr, gxc, gyc, gzc,
                    bvx, bvy, bvz, bnx, bny, bnz, ba, bone,
                    sir_v, sic_v, sem, *accs):
    cid = lax.axis_index("c")
    sid = lax.axis_index("s")
    wid = sid * NC + cid
    base_w = wid * EPW

    pltpu.sync_copy(row_hbm.at[pl.ds(base_w, EPW)], r_v)
    pltpu.sync_copy(col_hbm.at[pl.ds(base_w, EPW)], c_v)
    pltpu.sync_copy(ty_hbm.at[pl.ds(base_w, EPW)], t_v)
    pltpu.sync_copy(at_hbm.at[pl.ds(base_w, EPW)], a_v)
    pltpu.sync_copy(q_hbm.at[pl.ds(base_w, EPW)], q_v)

    for k in range(NACC):
        @pl.when(sid == k)
        def _zero(k=k):
            pltpu.sync_copy(z_hbm, accs[k])

    ones = jnp.full((16,), 1.0, jnp.float32)
    for g in range(SCG):
        bone[pl.ds(g * 16, 16)] = ones
    plsc.subcore_barrier()

    def body(ch, carry):
        off = ch * SCC
        ri = r_v.at[pl.ds(off, SCC)]
        ci = c_v.at[pl.ds(off, SCC)]
        gs = [
            pltpu.async_copy(cx_hbm.at[ri], gxr, sem),
            pltpu.async_copy(cy_hbm.at[ri], gyr, sem),
            pltpu.async_copy(cz_hbm.at[ri], gzr, sem),
            pltpu.async_copy(cx_hbm.at[ci], gxc, sem),
            pltpu.async_copy(cy_hbm.at[ci], gyc, sem),
            pltpu.async_copy(cz_hbm.at[ci], gzc, sem),
        ]
        for d in gs:
            d.wait()
        for g in range(SCG):
            oe = off + g * 16
            r = r_v[pl.ds(oe, 16)]
            c = c_v[pl.ds(oe, 16)]
            tt = t_v[pl.ds(oe, 16)]
            av = a_v[pl.ds(oe, 16)]
            qv = q_v[pl.ds(oe, 16)]
            s16 = pl.ds(g * 16, 16)
            vx = qv * (gxr[s16] - gxc[s16])
            vy = qv * (gyr[s16] - gyc[s16])
            vz = qv * (gzr[s16] - gzc[s16])
            sir_v[s16] = r * NT + tt
            sic_v[s16] = c * NT + tt
            bvx[s16] = vx
            bvy[s16] = vy
            bvz[s16] = vz
            bnx[s16] = -vx
            bny[s16] = -vy
            bnz[s16] = -vz
            ba[s16] = av
        ds = [
            pltpu.async_copy(bvx, accs[0].at[sir_v], sem, add=True),
            pltpu.async_copy(bvy, accs[1].at[sir_v], sem, add=True),
            pltpu.async_copy(bvz, accs[2].at[sir_v], sem, add=True),
            pltpu.async_copy(ba, accs[3].at[sir_v], sem, add=True),
            pltpu.async_copy(bone, accs[4].at[sir_v], sem, add=True),
            pltpu.async_copy(bnx, accs[0].at[sic_v], sem, add=True),
            pltpu.async_copy(bny, accs[1].at[sic_v], sem, add=True),
            pltpu.async_copy(bnz, accs[2].at[sic_v], sem, add=True),
            pltpu.async_copy(ba, accs[5].at[sic_v], sem, add=True),
            pltpu.async_copy(bone, accs[6].at[sic_v], sem, add=True),
        ]
        for d in ds:
            d.wait()
        return carry

    lax.fori_loop(0, SCH, body, 0)
    plsc.subcore_barrier()

    for k in range(NACC):
        @pl.when(sid == k)
        def _writeback(k=k):
            pltpu.sync_copy(accs[k],
                            out_hbm.at[pl.ds((cid * NACC + k) * NROWSP, NROWSP)])


def _scatter(row, col, ty, attr, q, cx, cy, cz, zeros):
    mesh = plsc.VectorSubcoreMesh(core_axis_name="c", subcore_axis_name="s",
                                  num_cores=NC, num_subcores=NS)
    fn = functools.partial(
        pl.kernel,
        out_type=jax.ShapeDtypeStruct((NC * NACC * NROWSP,), jnp.float32),
        mesh=mesh,
        scratch_types=[
            pltpu.VMEM((EPW,), jnp.int32),
            pltpu.VMEM((EPW,), jnp.int32),
            pltpu.VMEM((EPW,), jnp.int32),
            pltpu.VMEM((EPW,), jnp.float32),
            pltpu.VMEM((EPW,), jnp.float32),
            pltpu.VMEM((SCC,), jnp.float32),
            pltpu.VMEM((SCC,), jnp.float32),
            pltpu.VMEM((SCC,), jnp.float32),
            pltpu.VMEM((SCC,), jnp.float32),
            pltpu.VMEM((SCC,), jnp.float32),
            pltpu.VMEM((SCC,), jnp.float32),
            pltpu.VMEM((SCC,), jnp.float32),
            pltpu.VMEM((SCC,), jnp.float32),
            pltpu.VMEM((SCC,), jnp.float32),
            pltpu.VMEM((SCC,), jnp.float32),
            pltpu.VMEM((SCC,), jnp.float32),
            pltpu.VMEM((SCC,), jnp.float32),
            pltpu.VMEM((SCC,), jnp.float32),
            pltpu.VMEM((SCC,), jnp.float32),
            pltpu.VMEM((SCC,), jnp.int32),
            pltpu.VMEM((SCC,), jnp.int32),
            pltpu.SemaphoreType.DMA,
        ] + [pltpu.VMEM_SHARED((NROWSP,), jnp.float32) for _ in range(NACC)],
    )(_scatter_kernel)
    return fn(row, col, ty, attr, q, cx, cy, cz, zeros)


def _final_body(acc_ref, h_ref, coord_ref, wg1h_ref, wg1e_ref, bg1_ref,
                wg2_ref, bg2_ref, out_ref):
    acc = acc_ref[0] + acc_ref[1]
    logits = []
    eqs = []
    for t in range(NT):
        e0 = acc[3, :, t:t + 1] / jnp.maximum(acc[4, :, t:t + 1], 1.0)
        e1 = acc[5, :, t:t + 1] / jnp.maximum(acc[6, :, t:t + 1], 1.0)
        z = (jnp.dot(h_ref[:, t, :], wg1h_ref[:], preferred_element_type=jnp.float32)
             + e0 * wg1e_ref[0:1, :] + e1 * wg1e_ref[1:2, :] + bg1_ref[:])
        x = _silu(z)
        lg = jnp.sum(x * wg2_ref[:], axis=1, keepdims=True) + bg2_ref[0, 0]
        logits.append(lg)
        eqs.append(jnp.concatenate(
            [acc[0, :, t:t + 1], acc[1, :, t:t + 1], acc[2, :, t:t + 1]], axis=1))
    lg = jnp.concatenate(logits, axis=1)
    m = jnp.max(lg, axis=1, keepdims=True)
    w = jnp.exp(lg - m)
    w = w / jnp.sum(w, axis=1, keepdims=True)
    out = coord_ref[:]
    for t in range(NT):
        out = out + w[:, t:t + 1] * eqs[t]
    out_ref[:] = out


def _final(acc4, h, coord, wg1h, wg1e, bg1, wg2, bg2):
    n = h.shape[0]
    grid = n // NB
    full = lambda shape: pl.BlockSpec(shape, lambda i: tuple(0 for _ in shape))
    return pl.pallas_call(
        _final_body,
        grid=(grid,),
        in_specs=[
            pl.BlockSpec((NC, NACC, NB, NT), lambda i: (0, 0, i, 0)),
            pl.BlockSpec((NB, NT, HID), lambda i: (i, 0, 0)),
            pl.BlockSpec((NB, 3), lambda i: (i, 0)),
            full((HID, HID)),
            full((2, HID)),
            full((1, HID)),
            full((1, HID)),
            full((1, 1)),
        ],
        out_specs=pl.BlockSpec((NB, 3), lambda i: (i, 0)),
        out_shape=jax.ShapeDtypeStruct((n, 3), jnp.float32),
    )(acc4, h, coord, wg1h, wg1e, bg1, wg2, bg2)


def kernel(h, coord, edge_index, coord_diff, edge_attr, edge_mask, edge_length,
           N, params):
    names = ["bond", "angle", "torsion", "radius"]
    row = edge_index[0]
    col = edge_index[1]
    etype = jnp.argmax(edge_mask, axis=0).astype(jnp.int32)
    gr = row * NT + etype
    gc = col * NT + etype
    maskf = edge_mask.T.astype(jnp.float32)

    def stack(i):
        ws = jnp.stack([params["mlp_" + n][i][0] for n in names])
        bs = jnp.stack([params["mlp_" + n][i][1] for n in names])
        return ws, bs

    w1, b1 = stack(0)
    wa = w1[:, :HID, :]
    wb = w1[:, HID:2 * HID, :]
    w1c = w1[:, 2 * HID, :]
    w2, b2 = stack(1)
    w3, b3 = stack(2)
    w2 = w2.astype(jnp.bfloat16)
    w3 = w3.astype(jnp.bfloat16)
    w4, b4 = stack(3)
    w4 = w4[:, :, 0]
    b4 = b4.reshape(NT, 1)

    (wg1, bg1), (wg2, bg2) = params["w_gen"]
    wg1h = wg1[:HID, :]
    wg1e = wg1[HID:HID + 2, :]
    bg1 = bg1.reshape(1, HID)
    wg2 = wg2[:, 0].reshape(1, HID)
    bg2 = bg2.reshape(1, 1)

    a3, b3t = _proj(h, wa, wb)
    a2 = a3.reshape(NNODES * NT, D1)
    b2t = b3t.reshape(NNODES * NT, D1)

    ga, gb = _gather(a2, b2t, gr, gc)

    q = _edge_mlp(ga, gb, edge_attr, maskf, edge_length,
                  w1c, b1, w2, b2, w3, b3, w4, b4)

    zeros = jnp.zeros((NROWSP,), jnp.float32)
    acc = _scatter(row, col, etype, edge_attr[:, 0], q[:, 0],
                   coord[:, 0], coord[:, 1], coord[:, 2], zeros)
    acc4 = acc.reshape(NC, NACC, NROWSP)[:, :, :NROWS].reshape(
        NC, NACC, NNODES, NT)

    out = _final(acc4, h, coord, wg1h, wg1e, bg1, wg2, bg2)
    return out


# speedup vs baseline: 5.0261x; 1.0207x over previous
import functools

import jax
import jax.numpy as jnp
from jax import lax
from jax.experimental import pallas as pl
from jax.experimental.pallas import tpu as pltpu
from jax.experimental.pallas import tpu_sc as plsc

HID = 128
NT = 4
D1 = 2 * HID
NNODES = 10000
NEDGES = 320000

NC = 2
NS = 16
NW = NC * NS

EPW = NEDGES // NW
GK = 80
GCH = EPW // GK
SCC = 80
SCH = EPW // SCC
SCG = SCC // 16

NROWS = NNODES * NT
NROWSP = 40064
NACC = 7

EB = 1280
NB = 1000


def _silu(x):
    return x * (0.5 * jnp.tanh(0.5 * x) + 0.5)


def _proj_body(h_ref, wa_ref, wb_ref, a_ref, b_ref):
    for t in range(NT):
        ht = h_ref[:, t, :]
        a_ref[:, t, :] = jnp.dot(ht, wa_ref[t], preferred_element_type=jnp.float32)
        b_ref[:, t, :] = jnp.dot(ht, wb_ref[t], preferred_element_type=jnp.float32)


def _proj(h, wa, wb):
    n = h.shape[0]
    grid = n // NB
    return pl.pallas_call(
        _proj_body,
        grid=(grid,),
        in_specs=[
            pl.BlockSpec((NB, NT, HID), lambda i: (i, 0, 0)),
            pl.BlockSpec((NT, HID, D1), lambda i: (0, 0, 0)),
            pl.BlockSpec((NT, HID, D1), lambda i: (0, 0, 0)),
        ],
        out_specs=[
            pl.BlockSpec((NB, NT, D1), lambda i: (i, 0, 0)),
            pl.BlockSpec((NB, NT, D1), lambda i: (i, 0, 0)),
        ],
        out_shape=[
            jax.ShapeDtypeStruct((n, NT, D1), jnp.float32),
            jax.ShapeDtypeStruct((n, NT, D1), jnp.float32),
        ],
    )(h, wa, wb)


def _gather_kernel(a_hbm, b_hbm, gr_hbm, gc_hbm, g_hbm,
                   ir_v, ic_v, ra_v, sa, sb):
    cid = lax.axis_index("c")
    sid = lax.axis_index("s")
    wid = sid * NC + cid
    base_w = wid * EPW

    def body(i, carry):
        base = base_w + i * GK
        pltpu.sync_copy(gr_hbm.at[pl.ds(base, GK)], ir_v)
        pltpu.sync_copy(gc_hbm.at[pl.ds(base, GK)], ic_v)
        pltpu.async_copy(a_hbm.at[ir_v], ra_v, sa).wait()
        pltpu.async_copy(b_hbm.at[ic_v], ra_v, sb, add=True).wait()
        pltpu.sync_copy(ra_v, g_hbm.at[pl.ds(base, GK)])
        return carry

    lax.fori_loop(0, GCH, body, 0)


def _gather(a2, b2, gr, gc):
    mesh = plsc.VectorSubcoreMesh(core_axis_name="c", subcore_axis_name="s",
                                  num_cores=NC, num_subcores=NS)
    fn = functools.partial(
        pl.kernel,
        out_type=jax.ShapeDtypeStruct((NEDGES, D1), jnp.float32),
        mesh=mesh,
        scratch_types=[
            pltpu.VMEM((GK,), jnp.int32),
            pltpu.VMEM((GK,), jnp.int32),
            pltpu.VMEM((GK, D1), jnp.float32),
            pltpu.SemaphoreType.DMA,
            pltpu.SemaphoreType.DMA,
        ],
    )(_gather_kernel)
    return fn(a2, b2, gr, gc)


def _mlp_body(g_ref, attr_ref, mask_ref, el_ref,
              w1c_ref, b1_ref, w2_ref, b2_ref, w3_ref, b3_ref, w4_ref, b4_ref,
              q_ref):
    g = g_ref[:]
    a = attr_ref[:]
    m = mask_ref[:]
    w1cs = jnp.dot(m, w1c_ref[:], preferred_element_type=jnp.float32)
    b1s = jnp.dot(m, b1_ref[:], preferred_element_type=jnp.float32)
    x1 = _silu(g + a * w1cs + b1s).astype(jnp.bfloat16)
    l2 = jnp.dot(m, b2_ref[:], preferred_element_type=jnp.float32)
    for t in range(NT):
        l2 = l2 + m[:, t:t + 1] * jnp.dot(
            x1, w2_ref[t], preferred_element_type=jnp.float32)
    x2 = _silu(l2).astype(jnp.bfloat16)
    l3 = jnp.dot(m, b3_ref[:], preferred_element_type=jnp.float32)
    for t in range(NT):
        l3 = l3 + m[:, t:t + 1] * jnp.dot(
            x2, w3_ref[t], preferred_element_type=jnp.float32)
    x3 = _silu(l3)
    w4s = jnp.dot(m, w4_ref[:], preferred_element_type=jnp.float32)
    b4s = jnp.dot(m, b4_ref[:], preferred_element_type=jnp.float32)
    s = jnp.sum(x3 * w4s, axis=1, keepdims=True) + b4s
    q_ref[:] = s / el_ref[:]


def _edge_mlp(g, attr, maskf, el, w1c, b1, w2, b2, w3, b3, w4, b4):
    grid = NEDGES // EB
    full = lambda shape: pl.BlockSpec(shape, lambda i: tuple(0 for _ in shape))
    return pl.pallas_call(
        _mlp_body,
        grid=(grid,),
        in_specs=[
            pl.BlockSpec((EB, D1), lambda i: (i, 0)),
            pl.BlockSpec((EB, 1), lambda i: (i, 0)),
            pl.BlockSpec((EB, NT), lambda i: (i, 0)),
            pl.BlockSpec((EB, 1), lambda i: (i, 0)),
            full((NT, D1)),
            full((NT, D1)),
            full((NT, D1, HID)),
            full((NT, HID)),
            full((NT, HID, 64)),
            full((NT, 64)),
            full((NT, 64)),
            full((NT, 1)),
        ],
        out_specs=pl.BlockSpec((EB, 1), lambda i: (i, 0)),
        out_shape=jax.ShapeDtypeStruct((NEDGES, 1), jnp.float32),
    )(g, attr, maskf, el, w1c, b1, w2, b2, w3, b3, w4, b4)


def _scatter_kernel(row_hbm, col_hbm, ty_hbm, at_hbm, q_hbm,
                    cx_hbm, cy_hbm, cz_hbm, z_hbm, out_hbm,
                    r_v, c_v, t_v, a_v, q_v, gxr, gyr, gzr, gxc, gyc, gzc,
                    bvx, bvy, bvz, bnx, bny, bnz, ba, bone,
                    sir_v, sic_v, sem, *accs):
    cid = lax.axis_index("c")
    sid = lax.axis_index("s")
    wid = sid * NC + cid
    base_w = wid * EPW

    pltpu.sync_copy(row_hbm.at[pl.ds(base_w, EPW)], r_v)
    pltpu.sync_copy(col_hbm.at[pl.ds(base_w, EPW)], c_v)
    pltpu.sync_copy(ty_hbm.at[pl.ds(base_w, EPW)], t_v)
    pltpu.sync_copy(at_hbm.at[pl.ds(base_w, EPW)], a_v)
    pltpu.sync_copy(q_hbm.at[pl.ds(base_w, EPW)], q_v)

    for k in range(NACC):
        @pl.when(sid == k)
        def _zero(k=k):
            pltpu.sync_copy(z_hbm, accs[k])

    ones = jnp.full((16,), 1.0, jnp.float32)
    for g in range(SCG):
        bone[pl.ds(g * 16, 16)] = ones
    plsc.subcore_barrier()

    def body(ch, carry):
        off = ch * SCC
        ri = r_v.at[pl.ds(off, SCC)]
        ci = c_v.at[pl.ds(off, SCC)]
        gs = [
            pltpu.async_copy(cx_hbm.at[ri], gxr, sem),
            pltpu.async_copy(cy_hbm.at[ri], gyr, sem),
            pltpu.async_copy(cz_hbm.at[ri], gzr, sem),
            pltpu.async_copy(cx_hbm.at[ci], gxc, sem),
            pltpu.async_copy(cy_hbm.at[ci], gyc, sem),
            pltpu.async_copy(cz_hbm.at[ci], gzc, sem),
        ]
        for d in gs:
            d.wait()
        for g in range(SCG):
            oe = off + g * 16
            r = r_v[pl.ds(oe, 16)]
            c = c_v[pl.ds(oe, 16)]
            tt = t_v[pl.ds(oe, 16)]
            av = a_v[pl.ds(oe, 16)]
            qv = q_v[pl.ds(oe, 16)]
            s16 = pl.ds(g * 16, 16)
            vx = qv * (gxr[s16] - gxc[s16])
            vy = qv * (gyr[s16] - gyc[s16])
            vz = qv * (gzr[s16] - gzc[s16])
            sir_v[s16] = r * NT + tt
            sic_v[s16] = c * NT + tt
            bvx[s16] = vx
            bvy[s16] = vy
            bvz[s16] = vz
            bnx[s16] = -vx
            bny[s16] = -vy
            bnz[s16] = -vz
            ba[s16] = av
        ds = [
            pltpu.async_copy(bvx, accs[0].at[sir_v], sem, add=True),
            pltpu.async_copy(bvy, accs[1].at[sir_v], sem, add=True),
            pltpu.async_copy(bvz, accs[2].at[sir_v], sem, add=True),
            pltpu.async_copy(ba, accs[3].at[sir_v], sem, add=True),
            pltpu.async_copy(bone, accs[4].at[sir_v], sem, add=True),
            pltpu.async_copy(bnx, accs[0].at[sic_v], sem, add=True),
            pltpu.async_copy(bny, accs[1].at[sic_v], sem, add=True),
            pltpu.async_copy(bnz, accs[2].at[sic_v], sem, add=True),
            pltpu.async_copy(ba, accs[5].at[sic_v], sem, add=True),
            pltpu.async_copy(bone, accs[6].at[sic_v], sem, add=True),
        ]
        for d in ds:
            d.wait()
        return carry

    lax.fori_loop(0, SCH, body, 0)
    plsc.subcore_barrier()

    for k in range(NACC):
        @pl.when(sid == k)
        def _writeback(k=k):
            pltpu.sync_copy(accs[k],
                            out_hbm.at[pl.ds((cid * NACC + k) * NROWSP, NROWSP)])


def _scatter(row, col, ty, attr, q, cx, cy, cz, zeros):
    mesh = plsc.VectorSubcoreMesh(core_axis_name="c", subcore_axis_name="s",
                                  num_cores=NC, num_subcores=NS)
    fn = functools.partial(
        pl.kernel,
        out_type=jax.ShapeDtypeStruct((NC * NACC * NROWSP,), jnp.float32),
        mesh=mesh,
        scratch_types=[
            pltpu.VMEM((EPW,), jnp.int32),
            pltpu.VMEM((EPW,), jnp.int32),
            pltpu.VMEM((EPW,), jnp.int32),
            pltpu.VMEM((EPW,), jnp.float32),
            pltpu.VMEM((EPW,), jnp.float32),
            pltpu.VMEM((SCC,), jnp.float32),
            pltpu.VMEM((SCC,), jnp.float32),
            pltpu.VMEM((SCC,), jnp.float32),
            pltpu.VMEM((SCC,), jnp.float32),
            pltpu.VMEM((SCC,), jnp.float32),
            pltpu.VMEM((SCC,), jnp.float32),
            pltpu.VMEM((SCC,), jnp.float32),
            pltpu.VMEM((SCC,), jnp.float32),
            pltpu.VMEM((SCC,), jnp.float32),
            pltpu.VMEM((SCC,), jnp.float32),
            pltpu.VMEM((SCC,), jnp.float32),
            pltpu.VMEM((SCC,), jnp.float32),
            pltpu.VMEM((SCC,), jnp.float32),
            pltpu.VMEM((SCC,), jnp.float32),
            pltpu.VMEM((SCC,), jnp.int32),
            pltpu.VMEM((SCC,), jnp.int32),
            pltpu.SemaphoreType.DMA,
        ] + [pltpu.VMEM_SHARED((NROWSP,), jnp.float32) for _ in range(NACC)],
    )(_scatter_kernel)
    return fn(row, col, ty, attr, q, cx, cy, cz, zeros)


def _final_body(acc_ref, h_ref, coord_ref, wg1h_ref, wg1e_ref, bg1_ref,
                wg2_ref, bg2_ref, out_ref):
    acc = acc_ref[0] + acc_ref[1]
    logits = []
    eqs = []
    for t in range(NT):
        e0 = acc[3, :, t:t + 1] / jnp.maximum(acc[4, :, t:t + 1], 1.0)
        e1 = acc[5, :, t:t + 1] / jnp.maximum(acc[6, :, t:t + 1], 1.0)
        z = (jnp.dot(h_ref[:, t, :], wg1h_ref[:], preferred_element_type=jnp.float32)
             + e0 * wg1e_ref[0:1, :] + e1 * wg1e_ref[1:2, :] + bg1_ref[:])
        x = _silu(z)
        lg = jnp.sum(x * wg2_ref[:], axis=1, keepdims=True) + bg2_ref[0, 0]
        logits.append(lg)
        eqs.append(jnp.concatenate(
            [acc[0, :, t:t + 1], acc[1, :, t:t + 1], acc[2, :, t:t + 1]], axis=1))
    lg = jnp.concatenate(logits, axis=1)
    m = jnp.max(lg, axis=1, keepdims=True)
    w = jnp.exp(lg - m)
    w = w / jnp.sum(w, axis=1, keepdims=True)
    out = coord_ref[:]
    for t in range(NT):
        out = out + w[:, t:t + 1] * eqs[t]
    out_ref[:] = out


def _final(acc4, h, coord, wg1h, wg1e, bg1, wg2, bg2):
    n = h.shape[0]
    grid = n // NB
    full = lambda shape: pl.BlockSpec(shape, lambda i: tuple(0 for _ in shape))
    return pl.pallas_call(
        _final_body,
        grid=(grid,),
        in_specs=[
            pl.BlockSpec((NC, NACC, NB, NT), lambda i: (0, 0, i, 0)),
            pl.BlockSpec((NB, NT, HID), lambda i: (i, 0, 0)),
            pl.BlockSpec((NB, 3), lambda i: (i, 0)),
            full((HID, HID)),
            full((2, HID)),
            full((1, HID)),
            full((1, HID)),
            full((1, 1)),
        ],
        out_specs=pl.BlockSpec((NB, 3), lambda i: (i, 0)),
        out_shape=jax.ShapeDtypeStruct((n, 3), jnp.float32),
    )(acc4, h, coord, wg1h, wg1e, bg1, wg2, bg2)


def kernel(h, coord, edge_index, coord_diff, edge_attr, edge_mask, edge_length,
           N, params):
    names = ["bond", "angle", "torsion", "radius"]
    row = edge_index[0]
    col = edge_index[1]
    etype = jnp.argmax(edge_mask, axis=0).astype(jnp.int32)
    gr = row * NT + etype
    gc = col * NT + etype
    maskf = edge_mask.T.astype(jnp.float32)

    def stack(i):
        ws = jnp.stack([params["mlp_" + n][i][0] for n in names])
        bs = jnp.stack([params["mlp_" + n][i][1] for n in names])
        return ws, bs

    w1, b1 = stack(0)
    wa = w1[:, :HID, :]
    wb = w1[:, HID:2 * HID, :]
    w1c = w1[:, 2 * HID, :]
    w2, b2 = stack(1)
    w3, b3 = stack(2)
    w2 = w2.astype(jnp.bfloat16)
    w3 = w3.astype(jnp.bfloat16)
    w4, b4 = stack(3)
    w4 = w4[:, :, 0]
    b4 = b4.reshape(NT, 1)

    (wg1, bg1), (wg2, bg2) = params["w_gen"]
    wg1h = wg1[:HID, :]
    wg1e = wg1[HID:HID + 2, :]
    bg1 = bg1.reshape(1, HID)
    wg2 = wg2[:, 0].reshape(1, HID)
    bg2 = bg2.reshape(1, 1)

    a3, b3t = _proj(h, wa, wb)
    a2 = a3.reshape(NNODES * NT, D1)
    b2t = b3t.reshape(NNODES * NT, D1)

    g = _gather(a2, b2t, gr, gc)

    q = _edge_mlp(g, edge_attr, maskf, edge_length,
                  w1c, b1, w2, b2, w3, b3, w4, b4)

    zeros = jnp.zeros((NROWSP,), jnp.float32)
    acc = _scatter(row, col, etype, edge_attr[:, 0], q[:, 0],
                   coord[:, 0], coord[:, 1], coord[:, 2], zeros)
    acc4 = acc.reshape(NC, NACC, NROWSP)[:, :, :NROWS].reshape(
        NC, NACC, NNODES, NT)

    out = _final(acc4, h, coord, wg1h, wg1e, bg1, wg2, bg2)
    return out


# speedup vs baseline: 5.0907x; 1.0129x over previous
import functools

import jax
import jax.numpy as jnp
from jax import lax
from jax.experimental import pallas as pl
from jax.experimental.pallas import tpu as pltpu
from jax.experimental.pallas import tpu_sc as plsc

HID = 128
NT = 4
D1 = 2 * HID
NNODES = 10000
NEDGES = 320000

NC = 2
NS = 16
NW = NC * NS

GK = 80
SCC = 80
SCG = SCC // 16
SEGS = (158720, 161280)

NROWS = NNODES * NT
NROWSP = 40064
NACC = 7

EB = 1280
NB = 1000


def _silu(x):
    return x * (0.5 * jnp.tanh(0.5 * x) + 0.5)


def _proj_body(h_ref, wa_ref, wb_ref, a_ref, b_ref):
    for t in range(NT):
        ht = h_ref[:, t, :]
        a_ref[:, t, :] = jnp.dot(ht, wa_ref[t], preferred_element_type=jnp.float32)
        b_ref[:, t, :] = jnp.dot(ht, wb_ref[t], preferred_element_type=jnp.float32)


def _proj(h, wa, wb):
    n = h.shape[0]
    grid = n // NB
    return pl.pallas_call(
        _proj_body,
        grid=(grid,),
        in_specs=[
            pl.BlockSpec((NB, NT, HID), lambda i: (i, 0, 0)),
            pl.BlockSpec((NT, HID, D1), lambda i: (0, 0, 0)),
            pl.BlockSpec((NT, HID, D1), lambda i: (0, 0, 0)),
        ],
        out_specs=[
            pl.BlockSpec((NB, NT, D1), lambda i: (i, 0, 0)),
            pl.BlockSpec((NB, NT, D1), lambda i: (i, 0, 0)),
        ],
        out_shape=[
            jax.ShapeDtypeStruct((n, NT, D1), jnp.float32),
            jax.ShapeDtypeStruct((n, NT, D1), jnp.float32),
        ],
    )(h, wa, wb)


def _gather(a2, b2, gr, gc):
    n_e = gr.shape[0]
    epw = n_e // NW
    gch = epw // GK

    def _gather_kernel(a_hbm, b_hbm, gr_hbm, gc_hbm, ga_hbm, gb_hbm,
                       ir_v, ic_v, ra_v, rb_v, sa, sb):
        cid = lax.axis_index("c")
        sid = lax.axis_index("s")
        wid = sid * NC + cid
        base_w = wid * epw

        def body(i, carry):
            base = base_w + i * GK
            pltpu.sync_copy(gr_hbm.at[pl.ds(base, GK)], ir_v)
            pltpu.sync_copy(gc_hbm.at[pl.ds(base, GK)], ic_v)
            da = pltpu.async_copy(a_hbm.at[ir_v], ra_v, sa)
            db = pltpu.async_copy(b_hbm.at[ic_v], rb_v, sb)
            da.wait()
            db.wait()
            pltpu.sync_copy(ra_v, ga_hbm.at[pl.ds(base, GK)])
            pltpu.sync_copy(rb_v, gb_hbm.at[pl.ds(base, GK)])
            return carry

        lax.fori_loop(0, gch, body, 0)

    mesh = plsc.VectorSubcoreMesh(core_axis_name="c", subcore_axis_name="s",
                                  num_cores=NC, num_subcores=NS)
    fn = functools.partial(
        pl.kernel,
        out_type=(jax.ShapeDtypeStruct((n_e, D1), jnp.float32),
                  jax.ShapeDtypeStruct((n_e, D1), jnp.float32)),
        mesh=mesh,
        scratch_types=[
            pltpu.VMEM((GK,), jnp.int32),
            pltpu.VMEM((GK,), jnp.int32),
            pltpu.VMEM((GK, D1), jnp.float32),
            pltpu.VMEM((GK, D1), jnp.float32),
            pltpu.SemaphoreType.DMA,
            pltpu.SemaphoreType.DMA,
        ],
    )(_gather_kernel)
    return fn(a2, b2, gr, gc)


def _mlp_body(ga_ref, gb_ref, attr_ref, mask_ref, el_ref,
              w1c_ref, b1_ref, w2_ref, b2_ref, w3_ref, b3_ref, w4_ref, b4_ref,
              q_ref):
    g = ga_ref[:] + gb_ref[:]
    a = attr_ref[:]
    m = mask_ref[:]
    w1cs = jnp.dot(m, w1c_ref[:], preferred_element_type=jnp.float32)
    b1s = jnp.dot(m, b1_ref[:], preferred_element_type=jnp.float32)
    x1 = _silu(g + a * w1cs + b1s).astype(jnp.bfloat16)
    l2 = jnp.dot(m, b2_ref[:], preferred_element_type=jnp.float32)
    for t in range(NT):
        l2 = l2 + m[:, t:t + 1] * jnp.dot(
            x1, w2_ref[t], preferred_element_type=jnp.float32)
    x2 = _silu(l2).astype(jnp.bfloat16)
    l3 = jnp.dot(m, b3_ref[:], preferred_element_type=jnp.float32)
    for t in range(NT):
        l3 = l3 + m[:, t:t + 1] * jnp.dot(
            x2, w3_ref[t], preferred_element_type=jnp.float32)
    x3 = _silu(l3)
    w4s = jnp.dot(m, w4_ref[:], preferred_element_type=jnp.float32)
    b4s = jnp.dot(m, b4_ref[:], preferred_element_type=jnp.float32)
    s = jnp.sum(x3 * w4s, axis=1, keepdims=True) + b4s
    q_ref[:] = s / el_ref[:]


def _edge_mlp(ga, gb, attr, maskf, el, w1c, b1, w2, b2, w3, b3, w4, b4):
    n_e = ga.shape[0]
    grid = n_e // EB
    full = lambda shape: pl.BlockSpec(shape, lambda i: tuple(0 for _ in shape))
    return pl.pallas_call(
        _mlp_body,
        grid=(grid,),
        in_specs=[
            pl.BlockSpec((EB, D1), lambda i: (i, 0)),
            pl.BlockSpec((EB, D1), lambda i: (i, 0)),
            pl.BlockSpec((EB, 1), lambda i: (i, 0)),
            pl.BlockSpec((EB, NT), lambda i: (i, 0)),
            pl.BlockSpec((EB, 1), lambda i: (i, 0)),
            full((NT, D1)),
            full((NT, D1)),
            full((NT, D1, HID)),
            full((NT, HID)),
            full((NT, HID, 64)),
            full((NT, 64)),
            full((NT, 64)),
            full((NT, 1)),
        ],
        out_specs=pl.BlockSpec((EB, 1), lambda i: (i, 0)),
        out_shape=jax.ShapeDtypeStruct((n_e, 1), jnp.float32),
    )(ga, gb, attr, maskf, el, w1c, b1, w2, b2, w3, b3, w4, b4)


def _make_scatter_kernel(epw, sch):
  def _scatter_kernel(row_hbm, col_hbm, ty_hbm, at_hbm, q_hbm,
                      cx_hbm, cy_hbm, cz_hbm, z_hbm, out_hbm,
                      r_v, c_v, t_v, a_v, q_v, gxr, gyr, gzr, gxc, gyc, gzc,
                      bvx, bvy, bvz, bnx, bny, bnz, ba, bone,
                      sir_v, sic_v, sem, *accs):
    cid = lax.axis_index("c")
    sid = lax.axis_index("s")
    wid = sid * NC + cid
    base_w = wid * epw

    pltpu.sync_copy(row_hbm.at[pl.ds(base_w, epw)], r_v)
    pltpu.sync_copy(col_hbm.at[pl.ds(base_w, epw)], c_v)
    pltpu.sync_copy(ty_hbm.at[pl.ds(base_w, epw)], t_v)
    pltpu.sync_copy(at_hbm.at[pl.ds(base_w, epw)], a_v)
    pltpu.sync_copy(q_hbm.at[pl.ds(base_w, epw)], q_v)

    for k in range(NACC):
        @pl.when(sid == k)
        def _zero(k=k):
            pltpu.sync_copy(z_hbm, accs[k])

    ones = jnp.full((16,), 1.0, jnp.float32)
    for g in range(SCG):
        bone[pl.ds(g * 16, 16)] = ones
    plsc.subcore_barrier()

    def body(ch, carry):
        off = ch * SCC
        ri = r_v.at[pl.ds(off, SCC)]
        ci = c_v.at[pl.ds(off, SCC)]
        gs = [
            pltpu.async_copy(cx_hbm.at[ri], gxr, sem),
            pltpu.async_copy(cy_hbm.at[ri], gyr, sem),
            pltpu.async_copy(cz_hbm.at[ri], gzr, sem),
            pltpu.async_copy(cx_hbm.at[ci], gxc, sem),
            pltpu.async_copy(cy_hbm.at[ci], gyc, sem),
            pltpu.async_copy(cz_hbm.at[ci], gzc, sem),
        ]
        for d in gs:
            d.wait()
        for g in range(SCG):
            oe = off + g * 16
            r = r_v[pl.ds(oe, 16)]
            c = c_v[pl.ds(oe, 16)]
            tt = t_v[pl.ds(oe, 16)]
            av = a_v[pl.ds(oe, 16)]
            qv = q_v[pl.ds(oe, 16)]
            s16 = pl.ds(g * 16, 16)
            vx = qv * (gxr[s16] - gxc[s16])
            vy = qv * (gyr[s16] - gyc[s16])
            vz = qv * (gzr[s16] - gzc[s16])
            sir_v[s16] = r * NT + tt
            sic_v[s16] = c * NT + tt
            bvx[s16] = vx
            bvy[s16] = vy
            bvz[s16] = vz
            bnx[s16] = -vx
            bny[s16] = -vy
            bnz[s16] = -vz
            ba[s16] = av
        ds = [
            pltpu.async_copy(bvx, accs[0].at[sir_v], sem, add=True),
            pltpu.async_copy(bvy, accs[1].at[sir_v], sem, add=True),
            pltpu.async_copy(bvz, accs[2].at[sir_v], sem, add=True),
            pltpu.async_copy(ba, accs[3].at[sir_v], sem, add=True),
            pltpu.async_copy(bone, accs[4].at[sir_v], sem, add=True),
            pltpu.async_copy(bnx, accs[0].at[sic_v], sem, add=True),
            pltpu.async_copy(bny, accs[1].at[sic_v], sem, add=True),
            pltpu.async_copy(bnz, accs[2].at[sic_v], sem, add=True),
            pltpu.async_copy(ba, accs[5].at[sic_v], sem, add=True),
            pltpu.async_copy(bone, accs[6].at[sic_v], sem, add=True),
        ]
        for d in ds:
            d.wait()
        return carry

    lax.fori_loop(0, sch, body, 0)
    plsc.subcore_barrier()

    for k in range(NACC):
        @pl.when(sid == k)
        def _writeback(k=k):
            pltpu.sync_copy(accs[k],
                            out_hbm.at[pl.ds((cid * NACC + k) * NROWSP, NROWSP)])

  return _scatter_kernel


def _scatter(row, col, ty, attr, q, cx, cy, cz, zeros):
    n_e = row.shape[0]
    epw = n_e // NW
    sch = epw // SCC
    mesh = plsc.VectorSubcoreMesh(core_axis_name="c", subcore_axis_name="s",
                                  num_cores=NC, num_subcores=NS)
    fn = functools.partial(
        pl.kernel,
        out_type=jax.ShapeDtypeStruct((NC * NACC * NROWSP,), jnp.float32),
        mesh=mesh,
        scratch_types=[
            pltpu.VMEM((epw,), jnp.int32),
            pltpu.VMEM((epw,), jnp.int32),
            pltpu.VMEM((epw,), jnp.int32),
            pltpu.VMEM((epw,), jnp.float32),
            pltpu.VMEM((epw,), jnp.float32),
            pltpu.VMEM((SCC,), jnp.float32),
            pltpu.VMEM((SCC,), jnp.float32),
            pltpu.VMEM((SCC,), jnp.float32),
            pltpu.VMEM((SCC,), jnp.float32),
            pltpu.VMEM((SCC,), jnp.float32),
            pltpu.VMEM((SCC,), jnp.float32),
            pltpu.VMEM((SCC,), jnp.float32),
            pltpu.VMEM((SCC,), jnp.float32),
            pltpu.VMEM((SCC,), jnp.float32),
            pltpu.VMEM((SCC,), jnp.float32),
            pltpu.VMEM((SCC,), jnp.float32),
            pltpu.VMEM((SCC,), jnp.float32),
            pltpu.VMEM((SCC,), jnp.float32),
            pltpu.VMEM((SCC,), jnp.float32),
            pltpu.VMEM((SCC,), jnp.int32),
            pltpu.VMEM((SCC,), jnp.int32),
            pltpu.SemaphoreType.DMA,
        ] + [pltpu.VMEM_SHARED((NROWSP,), jnp.float32) for _ in range(NACC)],
    )(_make_scatter_kernel(epw, sch))
    return fn(row, col, ty, attr, q, cx, cy, cz, zeros)


def _final_body(acc_ref, accb_ref, h_ref, coord_ref, wg1h_ref, wg1e_ref, bg1_ref,
                wg2_ref, bg2_ref, out_ref):
    acc = (acc_ref[0] + acc_ref[1]
           + accb_ref[0] + accb_ref[1])
    logits = []
    eqs = []
    for t in range(NT):
        e0 = acc[3, :, t:t + 1] / jnp.maximum(acc[4, :, t:t + 1], 1.0)
        e1 = acc[5, :, t:t + 1] / jnp.maximum(acc[6, :, t:t + 1], 1.0)
        z = (jnp.dot(h_ref[:, t, :], wg1h_ref[:], preferred_element_type=jnp.float32)
             + e0 * wg1e_ref[0:1, :] + e1 * wg1e_ref[1:2, :] + bg1_ref[:])
        x = _silu(z)
        lg = jnp.sum(x * wg2_ref[:], axis=1, keepdims=True) + bg2_ref[0, 0]
        logits.append(lg)
        eqs.append(jnp.concatenate(
            [acc[0, :, t:t + 1], acc[1, :, t:t + 1], acc[2, :, t:t + 1]], axis=1))
    lg = jnp.concatenate(logits, axis=1)
    m = jnp.max(lg, axis=1, keepdims=True)
    w = jnp.exp(lg - m)
    w = w / jnp.sum(w, axis=1, keepdims=True)
    out = coord_ref[:]
    for t in range(NT):
        out = out + w[:, t:t + 1] * eqs[t]
    out_ref[:] = out


def _final(acc4, acc4b, h, coord, wg1h, wg1e, bg1, wg2, bg2):
    n = h.shape[0]
    grid = n // NB
    full = lambda shape: pl.BlockSpec(shape, lambda i: tuple(0 for _ in shape))
    return pl.pallas_call(
        _final_body,
        grid=(grid,),
        in_specs=[
            pl.BlockSpec((NC, NACC, NB, NT), lambda i: (0, 0, i, 0)),
            pl.BlockSpec((NC, NACC, NB, NT), lambda i: (0, 0, i, 0)),
            pl.BlockSpec((NB, NT, HID), lambda i: (i, 0, 0)),
            pl.BlockSpec((NB, 3), lambda i: (i, 0)),
            full((HID, HID)),
            full((2, HID)),
            full((1, HID)),
            full((1, HID)),
            full((1, 1)),
        ],
        out_specs=pl.BlockSpec((NB, 3), lambda i: (i, 0)),
        out_shape=jax.ShapeDtypeStruct((n, 3), jnp.float32),
    )(acc4, acc4b, h, coord, wg1h, wg1e, bg1, wg2, bg2)


def kernel(h, coord, edge_index, coord_diff, edge_attr, edge_mask, edge_length,
           N, params):
    names = ["bond", "angle", "torsion", "radius"]
    row = edge_index[0]
    col = edge_index[1]
    etype = jnp.argmax(edge_mask, axis=0).astype(jnp.int32)
    gr = row * NT + etype
    gc = col * NT + etype
    maskf = edge_mask.T.astype(jnp.float32)

    def stack(i):
        ws = jnp.stack([params["mlp_" + n][i][0] for n in names])
        bs = jnp.stack([params["mlp_" + n][i][1] for n in names])
        return ws, bs

    w1, b1 = stack(0)
    wa = w1[:, :HID, :]
    wb = w1[:, HID:2 * HID, :]
    w1c = w1[:, 2 * HID, :]
    w2, b2 = stack(1)
    w3, b3 = stack(2)
    w2 = w2.astype(jnp.bfloat16)
    w3 = w3.astype(jnp.bfloat16)
    w4, b4 = stack(3)
    w4 = w4[:, :, 0]
    b4 = b4.reshape(NT, 1)

    (wg1, bg1), (wg2, bg2) = params["w_gen"]
    wg1h = wg1[:HID, :]
    wg1e = wg1[HID:HID + 2, :]
    bg1 = bg1.reshape(1, HID)
    wg2 = wg2[:, 0].reshape(1, HID)
    bg2 = bg2.reshape(1, 1)

    a3, b3t = _proj(h, wa, wb)
    a2 = a3.reshape(NNODES * NT, D1)
    b2t = b3t.reshape(NNODES * NT, D1)

    zeros = jnp.zeros((NROWSP,), jnp.float32)
    cx, cy, cz = coord[:, 0], coord[:, 1], coord[:, 2]
    accs = []
    e0 = 0
    for n_e in SEGS:
        sl = slice(e0, e0 + n_e)
        ga, gb = _gather(a2, b2t, gr[sl], gc[sl])
        q = _edge_mlp(ga, gb, edge_attr[sl], maskf[sl], edge_length[sl],
                      w1c, b1, w2, b2, w3, b3, w4, b4)
        acc = _scatter(row[sl], col[sl], etype[sl], edge_attr[sl, 0], q[:, 0],
                       cx, cy, cz, zeros)
        accs.append(acc.reshape(NC, NACC, NROWSP)[:, :, :NROWS].reshape(
            NC, NACC, NNODES, NT))
        e0 += n_e

    out = _final(accs[0], accs[1], h, coord, wg1h, wg1e, bg1, wg2, bg2)
    return out


# speedup vs baseline: 5.2656x; 1.0344x over previous
import functools

import jax
import jax.numpy as jnp
from jax import lax
from jax.experimental import pallas as pl
from jax.experimental.pallas import tpu as pltpu
from jax.experimental.pallas import tpu_sc as plsc

HID = 128
NT = 4
D1 = 2 * HID
NNODES = 10000
NEDGES = 320000

NC = 2
NS = 16
NW = NC * NS

GK = 80
SCC = 80
SCG = SCC // 16
SEGS = (158720, 161280)

NROWS = NNODES * NT
NROWSP = 40064
NACC = 7

EB = 2560
NB = 1000


def _silu(x):
    return x * (0.5 * jnp.tanh(0.5 * x) + 0.5)


def _proj_body(h_ref, wa_ref, wb_ref, a_ref, b_ref):
    for t in range(NT):
        ht = h_ref[:, t, :]
        a_ref[:, t, :] = jnp.dot(ht, wa_ref[t], preferred_element_type=jnp.float32)
        b_ref[:, t, :] = jnp.dot(ht, wb_ref[t], preferred_element_type=jnp.float32)


def _proj(h, wa, wb):
    n = h.shape[0]
    grid = n // NB
    return pl.pallas_call(
        _proj_body,
        grid=(grid,),
        in_specs=[
            pl.BlockSpec((NB, NT, HID), lambda i: (i, 0, 0)),
            pl.BlockSpec((NT, HID, D1), lambda i: (0, 0, 0)),
            pl.BlockSpec((NT, HID, D1), lambda i: (0, 0, 0)),
        ],
        out_specs=[
            pl.BlockSpec((NB, NT, D1), lambda i: (i, 0, 0)),
            pl.BlockSpec((NB, NT, D1), lambda i: (i, 0, 0)),
        ],
        out_shape=[
            jax.ShapeDtypeStruct((n, NT, D1), jnp.float32),
            jax.ShapeDtypeStruct((n, NT, D1), jnp.float32),
        ],
    )(h, wa, wb)


def _gather(a2, b2, gr, gc):
    n_e = gr.shape[0]
    epw = n_e // NW
    gch = epw // GK

    def _gather_kernel(a_hbm, b_hbm, gr_hbm, gc_hbm, ga_hbm, gb_hbm,
                       ir_v, ic_v, ra_v, rb_v, sa, sb):
        cid = lax.axis_index("c")
        sid = lax.axis_index("s")
        wid = sid * NC + cid
        base_w = wid * epw

        def body(i, carry):
            base = base_w + i * GK
            pltpu.sync_copy(gr_hbm.at[pl.ds(base, GK)], ir_v)
            pltpu.sync_copy(gc_hbm.at[pl.ds(base, GK)], ic_v)
            da = pltpu.async_copy(a_hbm.at[ir_v], ra_v, sa)
            db = pltpu.async_copy(b_hbm.at[ic_v], rb_v, sb)
            da.wait()
            db.wait()
            pltpu.sync_copy(ra_v, ga_hbm.at[pl.ds(base, GK)])
            pltpu.sync_copy(rb_v, gb_hbm.at[pl.ds(base, GK)])
            return carry

        lax.fori_loop(0, gch, body, 0)

    mesh = plsc.VectorSubcoreMesh(core_axis_name="c", subcore_axis_name="s",
                                  num_cores=NC, num_subcores=NS)
    fn = functools.partial(
        pl.kernel,
        out_type=(jax.ShapeDtypeStruct((n_e, D1), jnp.float32),
                  jax.ShapeDtypeStruct((n_e, D1), jnp.float32)),
        mesh=mesh,
        scratch_types=[
            pltpu.VMEM((GK,), jnp.int32),
            pltpu.VMEM((GK,), jnp.int32),
            pltpu.VMEM((GK, D1), jnp.float32),
            pltpu.VMEM((GK, D1), jnp.float32),
            pltpu.SemaphoreType.DMA,
            pltpu.SemaphoreType.DMA,
        ],
    )(_gather_kernel)
    return fn(a2, b2, gr, gc)


def _mlp_body(ga_ref, gb_ref, attr_ref, mask_ref, el_ref,
              w1c_ref, b1_ref, w2_ref, b2_ref, w3_ref, b3_ref, w4_ref, b4_ref,
              q_ref):
    g = ga_ref[:] + gb_ref[:]
    a = attr_ref[:]
    m = mask_ref[:]
    w1cs = jnp.dot(m, w1c_ref[:], preferred_element_type=jnp.float32)
    b1s = jnp.dot(m, b1_ref[:], preferred_element_type=jnp.float32)
    x1 = _silu(g + a * w1cs + b1s).astype(jnp.bfloat16)
    l2 = jnp.dot(m, b2_ref[:], preferred_element_type=jnp.float32)
    for t in range(NT):
        l2 = l2 + m[:, t:t + 1] * jnp.dot(
            x1, w2_ref[t], preferred_element_type=jnp.float32)
    x2 = _silu(l2).astype(jnp.bfloat16)
    l3 = jnp.dot(m, b3_ref[:], preferred_element_type=jnp.float32)
    for t in range(NT):
        l3 = l3 + m[:, t:t + 1] * jnp.dot(
            x2, w3_ref[t], preferred_element_type=jnp.float32)
    x3 = _silu(l3)
    w4s = jnp.dot(m, w4_ref[:], preferred_element_type=jnp.float32)
    b4s = jnp.dot(m, b4_ref[:], preferred_element_type=jnp.float32)
    s = jnp.sum(x3 * w4s, axis=1, keepdims=True) + b4s
    q_ref[:] = s / el_ref[:]


def _edge_mlp(ga, gb, attr, maskf, el, w1c, b1, w2, b2, w3, b3, w4, b4):
    n_e = ga.shape[0]
    grid = n_e // EB
    full = lambda shape: pl.BlockSpec(shape, lambda i: tuple(0 for _ in shape))
    return pl.pallas_call(
        _mlp_body,
        grid=(grid,),
        in_specs=[
            pl.BlockSpec((EB, D1), lambda i: (i, 0)),
            pl.BlockSpec((EB, D1), lambda i: (i, 0)),
            pl.BlockSpec((EB, 1), lambda i: (i, 0)),
            pl.BlockSpec((EB, NT), lambda i: (i, 0)),
            pl.BlockSpec((EB, 1), lambda i: (i, 0)),
            full((NT, D1)),
            full((NT, D1)),
            full((NT, D1, HID)),
            full((NT, HID)),
            full((NT, HID, 64)),
            full((NT, 64)),
            full((NT, 64)),
            full((NT, 1)),
        ],
        out_specs=pl.BlockSpec((EB, 1), lambda i: (i, 0)),
        out_shape=jax.ShapeDtypeStruct((n_e, 1), jnp.float32),
    )(ga, gb, attr, maskf, el, w1c, b1, w2, b2, w3, b3, w4, b4)


def _make_scatter_kernel(epw, sch):
  def _scatter_kernel(row_hbm, col_hbm, ty_hbm, at_hbm, q_hbm,
                      cx_hbm, cy_hbm, cz_hbm, z_hbm, out_hbm,
                      r_v, c_v, t_v, a_v, q_v, gxr, gyr, gzr, gxc, gyc, gzc,
                      bvx, bvy, bvz, bnx, bny, bnz, ba, bone,
                      sir_v, sic_v, sem, *accs):
    cid = lax.axis_index("c")
    sid = lax.axis_index("s")
    wid = sid * NC + cid
    base_w = wid * epw

    pltpu.sync_copy(row_hbm.at[pl.ds(base_w, epw)], r_v)
    pltpu.sync_copy(col_hbm.at[pl.ds(base_w, epw)], c_v)
    pltpu.sync_copy(ty_hbm.at[pl.ds(base_w, epw)], t_v)
    pltpu.sync_copy(at_hbm.at[pl.ds(base_w, epw)], a_v)
    pltpu.sync_copy(q_hbm.at[pl.ds(base_w, epw)], q_v)

    for k in range(NACC):
        @pl.when(sid == k)
        def _zero(k=k):
            pltpu.sync_copy(z_hbm, accs[k])

    ones = jnp.full((16,), 1.0, jnp.float32)
    for g in range(SCG):
        bone[pl.ds(g * 16, 16)] = ones
    plsc.subcore_barrier()

    def body(ch, carry):
        off = ch * SCC
        ri = r_v.at[pl.ds(off, SCC)]
        ci = c_v.at[pl.ds(off, SCC)]
        gs = [
            pltpu.async_copy(cx_hbm.at[ri], gxr, sem),
            pltpu.async_copy(cy_hbm.at[ri], gyr, sem),
            pltpu.async_copy(cz_hbm.at[ri], gzr, sem),
            pltpu.async_copy(cx_hbm.at[ci], gxc, sem),
            pltpu.async_copy(cy_hbm.at[ci], gyc, sem),
            pltpu.async_copy(cz_hbm.at[ci], gzc, sem),
        ]
        for d in gs:
            d.wait()
        for g in range(SCG):
            oe = off + g * 16
            r = r_v[pl.ds(oe, 16)]
            c = c_v[pl.ds(oe, 16)]
            tt = t_v[pl.ds(oe, 16)]
            av = a_v[pl.ds(oe, 16)]
            qv = q_v[pl.ds(oe, 16)]
            s16 = pl.ds(g * 16, 16)
            vx = qv * (gxr[s16] - gxc[s16])
            vy = qv * (gyr[s16] - gyc[s16])
            vz = qv * (gzr[s16] - gzc[s16])
            sir_v[s16] = r * NT + tt
            sic_v[s16] = c * NT + tt
            bvx[s16] = vx
            bvy[s16] = vy
            bvz[s16] = vz
            bnx[s16] = -vx
            bny[s16] = -vy
            bnz[s16] = -vz
            ba[s16] = av
        ds = [
            pltpu.async_copy(bvx, accs[0].at[sir_v], sem, add=True),
            pltpu.async_copy(bvy, accs[1].at[sir_v], sem, add=True),
            pltpu.async_copy(bvz, accs[2].at[sir_v], sem, add=True),
            pltpu.async_copy(ba, accs[3].at[sir_v], sem, add=True),
            pltpu.async_copy(bone, accs[4].at[sir_v], sem, add=True),
            pltpu.async_copy(bnx, accs[0].at[sic_v], sem, add=True),
            pltpu.async_copy(bny, accs[1].at[sic_v], sem, add=True),
            pltpu.async_copy(bnz, accs[2].at[sic_v], sem, add=True),
            pltpu.async_copy(ba, accs[5].at[sic_v], sem, add=True),
            pltpu.async_copy(bone, accs[6].at[sic_v], sem, add=True),
        ]
        for d in ds:
            d.wait()
        return carry

    lax.fori_loop(0, sch, body, 0)
    plsc.subcore_barrier()

    for k in range(NACC):
        @pl.when(sid == k)
        def _writeback(k=k):
            pltpu.sync_copy(accs[k],
                            out_hbm.at[pl.ds((cid * NACC + k) * NROWSP, NROWSP)])

  return _scatter_kernel


def _scatter(row, col, ty, attr, q, cx, cy, cz, zeros):
    n_e = row.shape[0]
    epw = n_e // NW
    sch = epw // SCC
    mesh = plsc.VectorSubcoreMesh(core_axis_name="c", subcore_axis_name="s",
                                  num_cores=NC, num_subcores=NS)
    fn = functools.partial(
        pl.kernel,
        out_type=jax.ShapeDtypeStruct((NC * NACC * NROWSP,), jnp.float32),
        mesh=mesh,
        scratch_types=[
            pltpu.VMEM((epw,), jnp.int32),
            pltpu.VMEM((epw,), jnp.int32),
            pltpu.VMEM((epw,), jnp.int32),
            pltpu.VMEM((epw,), jnp.float32),
            pltpu.VMEM((epw,), jnp.float32),
            pltpu.VMEM((SCC,), jnp.float32),
            pltpu.VMEM((SCC,), jnp.float32),
            pltpu.VMEM((SCC,), jnp.float32),
            pltpu.VMEM((SCC,), jnp.float32),
            pltpu.VMEM((SCC,), jnp.float32),
            pltpu.VMEM((SCC,), jnp.float32),
            pltpu.VMEM((SCC,), jnp.float32),
            pltpu.VMEM((SCC,), jnp.float32),
            pltpu.VMEM((SCC,), jnp.float32),
            pltpu.VMEM((SCC,), jnp.float32),
            pltpu.VMEM((SCC,), jnp.float32),
            pltpu.VMEM((SCC,), jnp.float32),
            pltpu.VMEM((SCC,), jnp.float32),
            pltpu.VMEM((SCC,), jnp.float32),
            pltpu.VMEM((SCC,), jnp.int32),
            pltpu.VMEM((SCC,), jnp.int32),
            pltpu.SemaphoreType.DMA,
        ] + [pltpu.VMEM_SHARED((NROWSP,), jnp.float32) for _ in range(NACC)],
    )(_make_scatter_kernel(epw, sch))
    return fn(row, col, ty, attr, q, cx, cy, cz, zeros)


def _final_body(acc_ref, accb_ref, h_ref, coord_ref, wg1h_ref, wg1e_ref, bg1_ref,
                wg2_ref, bg2_ref, out_ref):
    acc = (acc_ref[0] + acc_ref[1]
           + accb_ref[0] + accb_ref[1])
    logits = []
    eqs = []
    for t in range(NT):
        e0 = acc[3, :, t:t + 1] / jnp.maximum(acc[4, :, t:t + 1], 1.0)
        e1 = acc[5, :, t:t + 1] / jnp.maximum(acc[6, :, t:t + 1], 1.0)
        z = (jnp.dot(h_ref[:, t, :], wg1h_ref[:], preferred_element_type=jnp.float32)
             + e0 * wg1e_ref[0:1, :] + e1 * wg1e_ref[1:2, :] + bg1_ref[:])
        x = _silu(z)
        lg = jnp.sum(x * wg2_ref[:], axis=1, keepdims=True) + bg2_ref[0, 0]
        logits.append(lg)
        eqs.append(jnp.concatenate(
            [acc[0, :, t:t + 1], acc[1, :, t:t + 1], acc[2, :, t:t + 1]], axis=1))
    lg = jnp.concatenate(logits, axis=1)
    m = jnp.max(lg, axis=1, keepdims=True)
    w = jnp.exp(lg - m)
    w = w / jnp.sum(w, axis=1, keepdims=True)
    out = coord_ref[:]
    for t in range(NT):
        out = out + w[:, t:t + 1] * eqs[t]
    out_ref[:] = out


def _final(acc4, acc4b, h, coord, wg1h, wg1e, bg1, wg2, bg2):
    n = h.shape[0]
    grid = n // NB
    full = lambda shape: pl.BlockSpec(shape, lambda i: tuple(0 for _ in shape))
    return pl.pallas_call(
        _final_body,
        grid=(grid,),
        in_specs=[
            pl.BlockSpec((NC, NACC, NB, NT), lambda i: (0, 0, i, 0)),
            pl.BlockSpec((NC, NACC, NB, NT), lambda i: (0, 0, i, 0)),
            pl.BlockSpec((NB, NT, HID), lambda i: (i, 0, 0)),
            pl.BlockSpec((NB, 3), lambda i: (i, 0)),
            full((HID, HID)),
            full((2, HID)),
            full((1, HID)),
            full((1, HID)),
            full((1, 1)),
        ],
        out_specs=pl.BlockSpec((NB, 3), lambda i: (i, 0)),
        out_shape=jax.ShapeDtypeStruct((n, 3), jnp.float32),
    )(acc4, acc4b, h, coord, wg1h, wg1e, bg1, wg2, bg2)


def kernel(h, coord, edge_index, coord_diff, edge_attr, edge_mask, edge_length,
           N, params):
    names = ["bond", "angle", "torsion", "radius"]
    row = edge_index[0]
    col = edge_index[1]
    etype = jnp.argmax(edge_mask, axis=0).astype(jnp.int32)
    gr = row * NT + etype
    gc = col * NT + etype
    maskf = edge_mask.T.astype(jnp.float32)

    def stack(i):
        ws = jnp.stack([params["mlp_" + n][i][0] for n in names])
        bs = jnp.stack([params["mlp_" + n][i][1] for n in names])
        return ws, bs

    w1, b1 = stack(0)
    wa = w1[:, :HID, :]
    wb = w1[:, HID:2 * HID, :]
    w1c = w1[:, 2 * HID, :]
    w2, b2 = stack(1)
    w3, b3 = stack(2)
    w2 = w2.astype(jnp.bfloat16)
    w3 = w3.astype(jnp.bfloat16)
    w4, b4 = stack(3)
    w4 = w4[:, :, 0]
    b4 = b4.reshape(NT, 1)

    (wg1, bg1), (wg2, bg2) = params["w_gen"]
    wg1h = wg1[:HID, :]
    wg1e = wg1[HID:HID + 2, :]
    bg1 = bg1.reshape(1, HID)
    wg2 = wg2[:, 0].reshape(1, HID)
    bg2 = bg2.reshape(1, 1)

    a3, b3t = _proj(h, wa, wb)
    a2 = a3.reshape(NNODES * NT, D1)
    b2t = b3t.reshape(NNODES * NT, D1)

    zeros = jnp.zeros((NROWSP,), jnp.float32)
    cx, cy, cz = coord[:, 0], coord[:, 1], coord[:, 2]
    accs = []
    e0 = 0
    for n_e in SEGS:
        sl = slice(e0, e0 + n_e)
        ga, gb = _gather(a2, b2t, gr[sl], gc[sl])
        q = _edge_mlp(ga, gb, edge_attr[sl], maskf[sl], edge_length[sl],
                      w1c, b1, w2, b2, w3, b3, w4, b4)
        acc = _scatter(row[sl], col[sl], etype[sl], edge_attr[sl, 0], q[:, 0],
                       cx, cy, cz, zeros)
        accs.append(acc.reshape(NC, NACC, NROWSP)[:, :, :NROWS].reshape(
            NC, NACC, NNODES, NT))
        e0 += n_e

    out = _final(accs[0], accs[1], h, coord, wg1h, wg1e, bg1, wg2, bg2)
    return out


# speedup vs baseline: 5.2687x; 1.0006x over previous
import functools

import jax
import jax.numpy as jnp
from jax import lax
from jax.experimental import pallas as pl
from jax.experimental.pallas import tpu as pltpu
from jax.experimental.pallas import tpu_sc as plsc

HID = 128
NT = 4
D1 = 2 * HID
NNODES = 10000
NEDGES = 320000

NC = 2
NS = 16
NW = NC * NS

GK = 80
SCC = 80
SCG = SCC // 16
SEGS = (158720, 161280)

NROWS = NNODES * NT
NROWSP = 40064
NACC = 7

EB = 2560
NB = 1000


def _silu(x):
    return x * (0.5 * jnp.tanh(0.5 * x) + 0.5)


def _proj_body(h_ref, wa_ref, wb_ref, a_ref, b_ref):
    for t in range(NT):
        ht = h_ref[:, t, :]
        a_ref[:, t, :] = jnp.dot(ht, wa_ref[t], preferred_element_type=jnp.float32)
        b_ref[:, t, :] = jnp.dot(ht, wb_ref[t], preferred_element_type=jnp.float32)


def _proj(h, wa, wb):
    n = h.shape[0]
    grid = n // NB
    return pl.pallas_call(
        _proj_body,
        grid=(grid,),
        in_specs=[
            pl.BlockSpec((NB, NT, HID), lambda i: (i, 0, 0)),
            pl.BlockSpec((NT, HID, D1), lambda i: (0, 0, 0)),
            pl.BlockSpec((NT, HID, D1), lambda i: (0, 0, 0)),
        ],
        out_specs=[
            pl.BlockSpec((NB, NT, D1), lambda i: (i, 0, 0)),
            pl.BlockSpec((NB, NT, D1), lambda i: (i, 0, 0)),
        ],
        out_shape=[
            jax.ShapeDtypeStruct((n, NT, D1), jnp.float32),
            jax.ShapeDtypeStruct((n, NT, D1), jnp.float32),
        ],
    )(h, wa, wb)


def _gather(a2, b2, gr, gc):
    n_e = gr.shape[0]
    epw = n_e // NW
    gch = epw // GK

    def _gather_kernel(a_hbm, b_hbm, gr_hbm, gc_hbm, ga_hbm, gb_hbm,
                       ir_v, ic_v, ra_v, rb_v, sa, sb):
        cid = lax.axis_index("c")
        sid = lax.axis_index("s")
        wid = sid * NC + cid
        base_w = wid * epw

        def body(i, carry):
            base = base_w + i * GK
            pltpu.sync_copy(gr_hbm.at[pl.ds(base, GK)], ir_v)
            pltpu.sync_copy(gc_hbm.at[pl.ds(base, GK)], ic_v)
            da = pltpu.async_copy(a_hbm.at[ir_v], ra_v, sa)
            db = pltpu.async_copy(b_hbm.at[ic_v], rb_v, sb)
            da.wait()
            db.wait()
            pltpu.sync_copy(ra_v, ga_hbm.at[pl.ds(base, GK)])
            pltpu.sync_copy(rb_v, gb_hbm.at[pl.ds(base, GK)])
            return carry

        lax.fori_loop(0, gch, body, 0)

    mesh = plsc.VectorSubcoreMesh(core_axis_name="c", subcore_axis_name="s",
                                  num_cores=NC, num_subcores=NS)
    fn = functools.partial(
        pl.kernel,
        out_type=(jax.ShapeDtypeStruct((n_e, D1), jnp.float32),
                  jax.ShapeDtypeStruct((n_e, D1), jnp.float32)),
        mesh=mesh,
        scratch_types=[
            pltpu.VMEM((GK,), jnp.int32),
            pltpu.VMEM((GK,), jnp.int32),
            pltpu.VMEM((GK, D1), jnp.float32),
            pltpu.VMEM((GK, D1), jnp.float32),
            pltpu.SemaphoreType.DMA,
            pltpu.SemaphoreType.DMA,
        ],
    )(_gather_kernel)
    return fn(a2, b2, gr, gc)


def _mlp_body(ga_ref, gb_ref, attr_ref, mask_ref, el_ref,
              w1c_ref, b1_ref, w2_ref, b2_ref, w3_ref, b3_ref, w4_ref, b4_ref,
              q_ref):
    g = ga_ref[:] + gb_ref[:]
    a = attr_ref[:]
    m = mask_ref[:]
    w1cs = jnp.dot(m, w1c_ref[:], preferred_element_type=jnp.float32)
    b1s = jnp.dot(m, b1_ref[:], preferred_element_type=jnp.float32)
    x1 = _silu(g + a * w1cs + b1s).astype(jnp.bfloat16)
    l2 = jnp.dot(m, b2_ref[:], preferred_element_type=jnp.float32)
    for t in range(NT):
        l2 = l2 + m[:, t:t + 1] * jnp.dot(
            x1, w2_ref[t], preferred_element_type=jnp.float32)
    x2 = _silu(l2).astype(jnp.bfloat16)
    l3 = jnp.dot(m, b3_ref[:], preferred_element_type=jnp.float32)
    for t in range(NT):
        l3 = l3 + m[:, t:t + 1] * jnp.dot(
            x2, w3_ref[t], preferred_element_type=jnp.float32)
    x3 = _silu(l3)
    w4s = jnp.dot(m, w4_ref[:], preferred_element_type=jnp.float32)
    b4s = jnp.dot(m, b4_ref[:], preferred_element_type=jnp.float32)
    s = jnp.sum(x3 * w4s, axis=1, keepdims=True) + b4s
    q_ref[:] = s / el_ref[:]


def _edge_mlp(ga, gb, attr, maskf, el, w1c, b1, w2, b2, w3, b3, w4, b4):
    n_e = ga.shape[0]
    grid = n_e // EB
    full = lambda shape: pl.BlockSpec(shape, lambda i: tuple(0 for _ in shape))
    return pl.pallas_call(
        _mlp_body,
        grid=(grid,),
        in_specs=[
            pl.BlockSpec((EB, D1), lambda i: (i, 0)),
            pl.BlockSpec((EB, D1), lambda i: (i, 0)),
            pl.BlockSpec((EB, 1), lambda i: (i, 0)),
            pl.BlockSpec((EB, NT), lambda i: (i, 0)),
            pl.BlockSpec((EB, 1), lambda i: (i, 0)),
            full((NT, D1)),
            full((NT, D1)),
            full((NT, D1, HID)),
            full((NT, HID)),
            full((NT, HID, 64)),
            full((NT, 64)),
            full((NT, 64)),
            full((NT, 1)),
        ],
        out_specs=pl.BlockSpec((EB, 1), lambda i: (i, 0)),
        out_shape=jax.ShapeDtypeStruct((n_e, 1), jnp.float32),
    )(ga, gb, attr, maskf, el, w1c, b1, w2, b2, w3, b3, w4, b4)


def _make_scatter_kernel(epw, sch):
  def _scatter_kernel(row_hbm, col_hbm, ty_hbm, at_hbm, q_hbm,
                      cx_hbm, cy_hbm, cz_hbm, z_hbm, out_hbm,
                      r_v, c_v, t_v, a_v, q_v, gxr, gyr, gzr, gxc, gyc, gzc,
                      bvx, bvy, bvz, bnx, bny, bnz, ba, bone,
                      sir_v, sic_v, sem, *accs):
    cid = lax.axis_index("c")
    sid = lax.axis_index("s")
    wid = sid * NC + cid
    base_w = wid * epw

    pltpu.sync_copy(row_hbm.at[pl.ds(base_w, epw)], r_v)
    pltpu.sync_copy(col_hbm.at[pl.ds(base_w, epw)], c_v)
    pltpu.sync_copy(ty_hbm.at[pl.ds(base_w, epw)], t_v)
    pltpu.sync_copy(at_hbm.at[pl.ds(base_w, epw)], a_v)
    pltpu.sync_copy(q_hbm.at[pl.ds(base_w, epw)], q_v)

    for k in range(NACC):
        @pl.when(sid == k)
        def _zero(k=k):
            pltpu.sync_copy(z_hbm, accs[k])

    ones = jnp.full((16,), 1.0, jnp.float32)
    for g in range(SCG):
        bone[pl.ds(g * 16, 16)] = ones
    plsc.subcore_barrier()

    def body(ch, carry):
        off = ch * SCC
        ri = r_v.at[pl.ds(off, SCC)]
        ci = c_v.at[pl.ds(off, SCC)]
        gs = [
            pltpu.async_copy(cx_hbm.at[ri], gxr, sem),
            pltpu.async_copy(cy_hbm.at[ri], gyr, sem),
            pltpu.async_copy(cz_hbm.at[ri], gzr, sem),
            pltpu.async_copy(cx_hbm.at[ci], gxc, sem),
            pltpu.async_copy(cy_hbm.at[ci], gyc, sem),
            pltpu.async_copy(cz_hbm.at[ci], gzc, sem),
        ]
        for d in gs:
            d.wait()
        for g in range(SCG):
            oe = off + g * 16
            r = r_v[pl.ds(oe, 16)]
            c = c_v[pl.ds(oe, 16)]
            tt = t_v[pl.ds(oe, 16)]
            av = a_v[pl.ds(oe, 16)]
            qv = q_v[pl.ds(oe, 16)]
            s16 = pl.ds(g * 16, 16)
            vx = qv * (gxr[s16] - gxc[s16])
            vy = qv * (gyr[s16] - gyc[s16])
            vz = qv * (gzr[s16] - gzc[s16])
            sir_v[s16] = r * NT + tt
            sic_v[s16] = c * NT + tt
            bvx[s16] = vx
            bvy[s16] = vy
            bvz[s16] = vz
            bnx[s16] = -vx
            bny[s16] = -vy
            bnz[s16] = -vz
            ba[s16] = av
        ds = [
            pltpu.async_copy(bvx, accs[0].at[sir_v], sem, add=True),
            pltpu.async_copy(bvy, accs[1].at[sir_v], sem, add=True),
            pltpu.async_copy(bvz, accs[2].at[sir_v], sem, add=True),
            pltpu.async_copy(ba, accs[3].at[sir_v], sem, add=True),
            pltpu.async_copy(bone, accs[4].at[sir_v], sem, add=True),
            pltpu.async_copy(bnx, accs[0].at[sic_v], sem, add=True),
            pltpu.async_copy(bny, accs[1].at[sic_v], sem, add=True),
            pltpu.async_copy(bnz, accs[2].at[sic_v], sem, add=True),
            pltpu.async_copy(ba, accs[5].at[sic_v], sem, add=True),
            pltpu.async_copy(bone, accs[6].at[sic_v], sem, add=True),
        ]
        for d in ds:
            d.wait()
        return carry

    lax.fori_loop(0, sch, body, 0)
    plsc.subcore_barrier()

    for k in range(NACC):
        @pl.when(sid == k)
        def _writeback(k=k):
            pltpu.sync_copy(accs[k],
                            out_hbm.at[pl.ds((cid * NACC + k) * NROWSP, NROWSP)])

  return _scatter_kernel


def _scatter(row, col, ty, attr, q, cx, cy, cz, zeros):
    n_e = row.shape[0]
    epw = n_e // NW
    sch = epw // SCC
    mesh = plsc.VectorSubcoreMesh(core_axis_name="c", subcore_axis_name="s",
                                  num_cores=NC, num_subcores=NS)
    fn = functools.partial(
        pl.kernel,
        out_type=jax.ShapeDtypeStruct((NC * NACC * NROWSP,), jnp.float32),
        mesh=mesh,
        scratch_types=[
            pltpu.VMEM((epw,), jnp.int32),
            pltpu.VMEM((epw,), jnp.int32),
            pltpu.VMEM((epw,), jnp.int32),
            pltpu.VMEM((epw,), jnp.float32),
            pltpu.VMEM((epw,), jnp.float32),
            pltpu.VMEM((SCC,), jnp.float32),
            pltpu.VMEM((SCC,), jnp.float32),
            pltpu.VMEM((SCC,), jnp.float32),
            pltpu.VMEM((SCC,), jnp.float32),
            pltpu.VMEM((SCC,), jnp.float32),
            pltpu.VMEM((SCC,), jnp.float32),
            pltpu.VMEM((SCC,), jnp.float32),
            pltpu.VMEM((SCC,), jnp.float32),
            pltpu.VMEM((SCC,), jnp.float32),
            pltpu.VMEM((SCC,), jnp.float32),
            pltpu.VMEM((SCC,), jnp.float32),
            pltpu.VMEM((SCC,), jnp.float32),
            pltpu.VMEM((SCC,), jnp.float32),
            pltpu.VMEM((SCC,), jnp.float32),
            pltpu.VMEM((SCC,), jnp.int32),
            pltpu.VMEM((SCC,), jnp.int32),
            pltpu.SemaphoreType.DMA,
        ] + [pltpu.VMEM_SHARED((NROWSP,), jnp.float32) for _ in range(NACC)],
    )(_make_scatter_kernel(epw, sch))
    return fn(row, col, ty, attr, q, cx, cy, cz, zeros)


def _final_body(*refs):
    acc_refs = refs[:len(SEGS)]
    (h_ref, coord_ref, wg1h_ref, wg1e_ref, bg1_ref,
     wg2_ref, bg2_ref, out_ref) = refs[len(SEGS):]
    acc = acc_refs[0][0] + acc_refs[0][1]
    for ar in acc_refs[1:]:
        acc = acc + ar[0] + ar[1]
    logits = []
    eqs = []
    for t in range(NT):
        e0 = acc[3, :, t:t + 1] / jnp.maximum(acc[4, :, t:t + 1], 1.0)
        e1 = acc[5, :, t:t + 1] / jnp.maximum(acc[6, :, t:t + 1], 1.0)
        z = (jnp.dot(h_ref[:, t, :], wg1h_ref[:], preferred_element_type=jnp.float32)
             + e0 * wg1e_ref[0:1, :] + e1 * wg1e_ref[1:2, :] + bg1_ref[:])
        x = _silu(z)
        lg = jnp.sum(x * wg2_ref[:], axis=1, keepdims=True) + bg2_ref[0, 0]
        logits.append(lg)
        eqs.append(jnp.concatenate(
            [acc[0, :, t:t + 1], acc[1, :, t:t + 1], acc[2, :, t:t + 1]], axis=1))
    lg = jnp.concatenate(logits, axis=1)
    m = jnp.max(lg, axis=1, keepdims=True)
    w = jnp.exp(lg - m)
    w = w / jnp.sum(w, axis=1, keepdims=True)
    out = coord_ref[:]
    for t in range(NT):
        out = out + w[:, t:t + 1] * eqs[t]
    out_ref[:] = out


def _final(accs, h, coord, wg1h, wg1e, bg1, wg2, bg2):
    n = h.shape[0]
    grid = n // NB
    full = lambda shape: pl.BlockSpec(shape, lambda i: tuple(0 for _ in shape))
    return pl.pallas_call(
        _final_body,
        grid=(grid,),
        in_specs=[
            pl.BlockSpec((NC, NACC, NB, NT), lambda i: (0, 0, i, 0))
            for _ in SEGS
        ] + [
            pl.BlockSpec((NB, NT, HID), lambda i: (i, 0, 0)),
            pl.BlockSpec((NB, 3), lambda i: (i, 0)),
            full((HID, HID)),
            full((2, HID)),
            full((1, HID)),
            full((1, HID)),
            full((1, 1)),
        ],
        out_specs=pl.BlockSpec((NB, 3), lambda i: (i, 0)),
        out_shape=jax.ShapeDtypeStruct((n, 3), jnp.float32),
    )(*accs, h, coord, wg1h, wg1e, bg1, wg2, bg2)


def kernel(h, coord, edge_index, coord_diff, edge_attr, edge_mask, edge_length,
           N, params):
    names = ["bond", "angle", "torsion", "radius"]
    row = edge_index[0]
    col = edge_index[1]
    etype = jnp.argmax(edge_mask, axis=0).astype(jnp.int32)
    gr = row * NT + etype
    gc = col * NT + etype
    maskf = edge_mask.T.astype(jnp.float32)

    def stack(i):
        ws = jnp.stack([params["mlp_" + n][i][0] for n in names])
        bs = jnp.stack([params["mlp_" + n][i][1] for n in names])
        return ws, bs

    w1, b1 = stack(0)
    wa = w1[:, :HID, :]
    wb = w1[:, HID:2 * HID, :]
    w1c = w1[:, 2 * HID, :]
    w2, b2 = stack(1)
    w3, b3 = stack(2)
    w2 = w2.astype(jnp.bfloat16)
    w3 = w3.astype(jnp.bfloat16)
    w4, b4 = stack(3)
    w4 = w4[:, :, 0]
    b4 = b4.reshape(NT, 1)

    (wg1, bg1), (wg2, bg2) = params["w_gen"]
    wg1h = wg1[:HID, :]
    wg1e = wg1[HID:HID + 2, :]
    bg1 = bg1.reshape(1, HID)
    wg2 = wg2[:, 0].reshape(1, HID)
    bg2 = bg2.reshape(1, 1)

    a3, b3t = _proj(h, wa, wb)
    a2 = a3.reshape(NNODES * NT, D1)
    b2t = b3t.reshape(NNODES * NT, D1)

    zeros = jnp.zeros((NROWSP,), jnp.float32)
    cx, cy, cz = coord[:, 0], coord[:, 1], coord[:, 2]
    accs = []
    e0 = 0
    for n_e in SEGS:
        sl = slice(e0, e0 + n_e)
        ga, gb = _gather(a2, b2t, gr[sl], gc[sl])
        q = _edge_mlp(ga, gb, edge_attr[sl], maskf[sl], edge_length[sl],
                      w1c, b1, w2, b2, w3, b3, w4, b4)
        acc = _scatter(row[sl], col[sl], etype[sl], edge_attr[sl, 0], q[:, 0],
                       cx, cy, cz, zeros)
        accs.append(acc.reshape(NC, NACC, NROWSP)[:, :, :NROWS].reshape(
            NC, NACC, NNODES, NT))
        e0 += n_e

    out = _final(accs, h, coord, wg1h, wg1e, bg1, wg2, bg2)
    return out
